# baseline (speedup 1.0000x reference)
import functools

import jax
import jax.numpy as jnp
from jax import lax
from jax.experimental import pallas as pl
from jax.experimental.pallas import tpu as pltpu
from jax.experimental.pallas import tpu_sc as plsc

N = 10000
E = 320000
CH = 128

NC = 2
NS = 16
NW = NC * NS

CHUNK = 128
EPW_CHUNKS = -(-E // (NW * CHUNK))
EPW = EPW_CHUNKS * CHUNK
E_PAD = EPW * NW
TOT_CHUNKS = E_PAD // CHUNK
CPT_A = 1
CPT_B = 157
N_PAD = 10112
RPT = N_PAD // NS
DUMMY = N

_mesh = plsc.VectorSubcoreMesh(core_axis_name="c", subcore_axis_name="s")
_sc_params = pltpu.CompilerParams(needs_layout_passes=False)


@functools.partial(
    pl.kernel,
    out_type=jax.ShapeDtypeStruct((NW, N_PAD), jnp.float32),
    mesh=_mesh,
    compiler_params=_sc_params,
    scratch_types=[
        pltpu.VMEM((EPW,), jnp.int32),
        pltpu.VMEM((N_PAD,), jnp.float32),
    ],
)
def _deg_kernel(dst_hbm, out_hbm, dstv, degv):
    wid = lax.axis_index("s") * NC + lax.axis_index("c")
    pltpu.sync_copy(dst_hbm.at[wid], dstv)

    zeros16 = jnp.zeros((16,), jnp.float32)
    ones16 = jnp.ones((16,), jnp.float32)

    def zero_body(i, _):
        degv[pl.ds(i * 16, 16)] = zeros16
        return 0

    lax.fori_loop(0, N_PAD // 16, zero_body, 0)

    def hist_body(i, _):
        idx = dstv[pl.ds(i * 16, 16)]
        plsc.addupdate_scatter(degv, [idx], ones16)
        return 0

    lax.fori_loop(0, EPW // 16, hist_body, 0)
    pltpu.sync_copy(degv, out_hbm.at[wid])


@functools.partial(
    pl.kernel,
    out_type=jax.ShapeDtypeStruct((NC, N_PAD, CH), jnp.float32),
    mesh=_mesh,
    compiler_params=_sc_params,
    scratch_types=[
        pltpu.VMEM((3, CHUNK), jnp.int32),
        pltpu.VMEM((3, CHUNK), jnp.int32),
        pltpu.VMEM((2, CHUNK, CH), jnp.float32),
        pltpu.VMEM_SHARED((N_PAD, CH), jnp.float32),
        pltpu.SemaphoreType.DMA,
        pltpu.SemaphoreType.DMA,
    ],
)
def _scatter_kernel(ya_hbm, yb_hbm, src_hbm, dst_hbm, acc_hbm,
                    sidx, didx, gbuf, acc_sh, semI, semG):
    c = lax.axis_index("c")
    s = lax.axis_index("s")
    off = jnp.where(c == 0, s * CPT_A, NS * CPT_A + s * CPT_B)
    nchunks = jnp.where(c == 0, CPT_A, CPT_B)

    pltpu.sync_copy(src_hbm.at[off], sidx.at[0])
    pltpu.sync_copy(dst_hbm.at[off], didx.at[0])

    @pl.when(c == 0)
    def _():
        pltpu.async_copy(ya_hbm.at[sidx.at[0]], gbuf.at[0], semG)

    @pl.when(c == 1)
    def _():
        pltpu.async_copy(yb_hbm.at[sidx.at[0]], gbuf.at[0], semG)

    @pl.when(nchunks > 1)
    def _():
        pltpu.async_copy(src_hbm.at[off + 1], sidx.at[1], semI)
        pltpu.async_copy(dst_hbm.at[off + 1], didx.at[1], semI)

    zeros16 = jnp.zeros((16,), jnp.float32)

    def zero_body(i, _):
        gbuf[1, i // (CH // 16), pl.ds((i % (CH // 16)) * 16, 16)] = zeros16
        return 0

    lax.fori_loop(0, CHUNK * CH // 16, zero_body, 0)

    row0 = s * RPT
    pltpu.sync_copy(gbuf.at[1], acc_sh.at[pl.ds(row0, CHUNK)])
    pltpu.sync_copy(gbuf.at[1], acc_sh.at[pl.ds(row0 + CHUNK, CHUNK)])
    pltpu.sync_copy(gbuf.at[1], acc_sh.at[pl.ds(row0 + 2 * CHUNK, CHUNK)])
    pltpu.sync_copy(gbuf.at[1], acc_sh.at[pl.ds(row0 + 3 * CHUNK, CHUNK)])
    pltpu.sync_copy(gbuf.at[1, pl.ds(0, RPT - 4 * CHUNK)],
                    acc_sh.at[pl.ds(row0 + 4 * CHUNK, RPT - 4 * CHUNK)])
    plsc.subcore_barrier()

    def _make_body(y_hbm):
        def body(j, _):
            cur = j % 2
            nxt = 1 - cur
            cur3 = j % 3
            pltpu.make_async_copy(y_hbm.at[sidx.at[cur3]], gbuf.at[cur],
                                  semG).wait()

            @pl.when(j + 1 < nchunks)
            def _():
                nxt3 = (j + 1) % 3
                pltpu.make_async_copy(src_hbm.at[off + j + 1], sidx.at[nxt3],
                                      semI).wait()
                pltpu.make_async_copy(dst_hbm.at[off + j + 1], didx.at[nxt3],
                                      semI).wait()
                pltpu.async_copy(y_hbm.at[sidx.at[nxt3]], gbuf.at[nxt], semG)

            pltpu.sync_copy(gbuf.at[cur], acc_sh.at[didx.at[cur3]], add=True)

            @pl.when(j + 2 < nchunks)
            def _():
                n2 = (j + 2) % 3
                pltpu.async_copy(src_hbm.at[off + j + 2], sidx.at[n2], semI)
                pltpu.async_copy(dst_hbm.at[off + j + 2], didx.at[n2], semI)

            return 0

        return body

    @pl.when(c == 0)
    def _():
        lax.fori_loop(0, CPT_A, _make_body(ya_hbm), 0)

    @pl.when(c == 1)
    def _():
        lax.fori_loop(0, CPT_B, _make_body(yb_hbm), 0)

    plsc.subcore_barrier()

    pltpu.sync_copy(acc_sh.at[pl.ds(row0, RPT)],
                    acc_hbm.at[c, pl.ds(row0, RPT)])


def _linear_body(x_ref, w_ref, deg_ref, y_ref, yb_ref):
    deg = jnp.sum(deg_ref[...], axis=1) + 1.0
    dinv = lax.rsqrt(deg)
    xw = jnp.dot(x_ref[...], w_ref[...], preferred_element_type=jnp.float32)
    y = dinv[:, None] * xw
    y_ref[...] = y
    yb_ref[...] = y


def _combine_body(acc_ref, y_ref, deg_ref, b_ref, o_ref):
    deg = jnp.sum(deg_ref[...], axis=1) + 1.0
    dinv = lax.rsqrt(deg)
    total = acc_ref[0] + acc_ref[1] + y_ref[...]
    o_ref[...] = dinv[:, None] * total + b_ref[...]


_ROWS_BLK = 1000


def kernel(x, edge_index, W, b):
    src = edge_index[0].astype(jnp.int32)
    dst = edge_index[1].astype(jnp.int32)
    pad = E_PAD - E
    src_p = jnp.concatenate([src, jnp.zeros((pad,), jnp.int32)])
    dst_pad = DUMMY + (jnp.arange(pad, dtype=jnp.int32) % (N_PAD - N))
    dst_p = jnp.concatenate([dst, dst_pad])

    deg_part = _deg_kernel(dst_p.reshape(NW, EPW)).T

    y, yb = pl.pallas_call(
        _linear_body,
        grid=(N // _ROWS_BLK,),
        in_specs=[
            pl.BlockSpec((_ROWS_BLK, CH), lambda i: (i, 0)),
            pl.BlockSpec((CH, CH), lambda i: (0, 0)),
            pl.BlockSpec((_ROWS_BLK, NW), lambda i: (i, 0)),
        ],
        out_specs=[
            pl.BlockSpec((_ROWS_BLK, CH), lambda i: (i, 0)),
            pl.BlockSpec((_ROWS_BLK, CH), lambda i: (i, 0)),
        ],
        out_shape=[
            jax.ShapeDtypeStruct((N, CH), jnp.float32),
            jax.ShapeDtypeStruct((N, CH), jnp.float32),
        ],
    )(x, W, deg_part)

    acc = _scatter_kernel(y, yb, src_p.reshape(TOT_CHUNKS, CHUNK),
                          dst_p.reshape(TOT_CHUNKS, CHUNK))

    out = pl.pallas_call(
        _combine_body,
        grid=(N // _ROWS_BLK,),
        in_specs=[
            pl.BlockSpec((NC, _ROWS_BLK, CH), lambda i: (0, i, 0)),
            pl.BlockSpec((_ROWS_BLK, CH), lambda i: (i, 0)),
            pl.BlockSpec((_ROWS_BLK, NW), lambda i: (i, 0)),
            pl.BlockSpec((1, CH), lambda i: (0, 0)),
        ],
        out_specs=pl.BlockSpec((_ROWS_BLK, CH), lambda i: (i, 0)),
        out_shape=jax.ShapeDtypeStruct((N, CH), jnp.float32),
    )(acc, y, deg_part, b.reshape(1, CH))

    return out


# speedup vs baseline: 1.1848x; 1.1848x over previous
import functools

import jax
import jax.numpy as jnp
from jax import lax
from jax.experimental import pallas as pl
from jax.experimental.pallas import tpu as pltpu
from jax.experimental.pallas import tpu_sc as plsc

N = 10000
E = 320000
CH = 128

NC = 2
NS = 16
NW = NC * NS

CHUNK = 128
EPW_CHUNKS = -(-E // (NW * CHUNK))
EPW = EPW_CHUNKS * CHUNK
E_PAD = EPW * NW
TOT_CHUNKS = E_PAD // CHUNK
CPT_A = EPW_CHUNKS
CPT_B = EPW_CHUNKS
N_PAD = 10112
RPT = N_PAD // NS
DUMMY = N

_mesh = plsc.VectorSubcoreMesh(core_axis_name="c", subcore_axis_name="s")
_sc_params = pltpu.CompilerParams(needs_layout_passes=False)


@functools.partial(
    pl.kernel,
    out_type=jax.ShapeDtypeStruct((NW, N_PAD), jnp.float32),
    mesh=_mesh,
    compiler_params=_sc_params,
    scratch_types=[
        pltpu.VMEM((EPW,), jnp.int32),
        pltpu.VMEM((N_PAD,), jnp.float32),
    ],
)
def _deg_kernel(dst_hbm, out_hbm, dstv, degv):
    wid = lax.axis_index("s") * NC + lax.axis_index("c")
    pltpu.sync_copy(dst_hbm.at[wid], dstv)

    zeros16 = jnp.zeros((16,), jnp.float32)
    ones16 = jnp.ones((16,), jnp.float32)

    def zero_body(i, _):
        degv[pl.ds(i * 16, 16)] = zeros16
        return 0

    lax.fori_loop(0, N_PAD // 16, zero_body, 0)

    def hist_body(i, _):
        idx = dstv[pl.ds(i * 16, 16)]
        plsc.addupdate_scatter(degv, [idx], ones16)
        return 0

    lax.fori_loop(0, EPW // 16, hist_body, 0)
    pltpu.sync_copy(degv, out_hbm.at[wid])


@functools.partial(
    pl.kernel,
    out_type=jax.ShapeDtypeStruct((NC, N_PAD, CH), jnp.float32),
    mesh=_mesh,
    compiler_params=_sc_params,
    scratch_types=[
        pltpu.VMEM((3, CHUNK), jnp.int32),
        pltpu.VMEM((3, CHUNK), jnp.int32),
        pltpu.VMEM((2, CHUNK, CH), jnp.float32),
        pltpu.VMEM_SHARED((N_PAD, CH), jnp.float32),
        pltpu.SemaphoreType.DMA,
        pltpu.SemaphoreType.DMA,
    ],
)
def _scatter_kernel(ya_hbm, yb_hbm, src_hbm, dst_hbm, acc_hbm,
                    sidx, didx, gbuf, acc_sh, semI, semG):
    c = lax.axis_index("c")
    s = lax.axis_index("s")
    off = jnp.where(c == 0, s * CPT_A, NS * CPT_A + s * CPT_B)
    nchunks = jnp.where(c == 0, CPT_A, CPT_B)

    pltpu.sync_copy(src_hbm.at[off], sidx.at[0])
    pltpu.sync_copy(dst_hbm.at[off], didx.at[0])

    pltpu.async_copy(ya_hbm.at[sidx.at[0]], gbuf.at[0], semG)

    @pl.when(nchunks > 1)
    def _():
        pltpu.async_copy(src_hbm.at[off + 1], sidx.at[1], semI)
        pltpu.async_copy(dst_hbm.at[off + 1], didx.at[1], semI)

    zeros16 = jnp.zeros((16,), jnp.float32)

    def zero_body(i, _):
        gbuf[1, i // (CH // 16), pl.ds((i % (CH // 16)) * 16, 16)] = zeros16
        return 0

    lax.fori_loop(0, CHUNK * CH // 16, zero_body, 0)

    row0 = s * RPT
    pltpu.sync_copy(gbuf.at[1], acc_sh.at[pl.ds(row0, CHUNK)])
    pltpu.sync_copy(gbuf.at[1], acc_sh.at[pl.ds(row0 + CHUNK, CHUNK)])
    pltpu.sync_copy(gbuf.at[1], acc_sh.at[pl.ds(row0 + 2 * CHUNK, CHUNK)])
    pltpu.sync_copy(gbuf.at[1], acc_sh.at[pl.ds(row0 + 3 * CHUNK, CHUNK)])
    pltpu.sync_copy(gbuf.at[1, pl.ds(0, RPT - 4 * CHUNK)],
                    acc_sh.at[pl.ds(row0 + 4 * CHUNK, RPT - 4 * CHUNK)])
    plsc.subcore_barrier()

    def body(j, _):
        cur = j % 2
        nxt = 1 - cur
        cur3 = j % 3

        @pl.when(cur == 0)
        def _():
            pltpu.make_async_copy(ya_hbm.at[sidx.at[cur3]], gbuf.at[cur],
                                  semG).wait()

        @pl.when(cur == 1)
        def _():
            pltpu.make_async_copy(yb_hbm.at[sidx.at[cur3]], gbuf.at[cur],
                                  semG).wait()

        @pl.when(j + 1 < nchunks)
        def _():
            nxt3 = (j + 1) % 3
            pltpu.make_async_copy(src_hbm.at[off + j + 1], sidx.at[nxt3],
                                  semI).wait()
            pltpu.make_async_copy(dst_hbm.at[off + j + 1], didx.at[nxt3],
                                  semI).wait()

            @pl.when(nxt == 0)
            def _():
                pltpu.async_copy(ya_hbm.at[sidx.at[nxt3]], gbuf.at[nxt], semG)

            @pl.when(nxt == 1)
            def _():
                pltpu.async_copy(yb_hbm.at[sidx.at[nxt3]], gbuf.at[nxt], semG)

        pltpu.sync_copy(gbuf.at[cur], acc_sh.at[didx.at[cur3]], add=True)

        @pl.when(j + 2 < nchunks)
        def _():
            n2 = (j + 2) % 3
            pltpu.async_copy(src_hbm.at[off + j + 2], sidx.at[n2], semI)
            pltpu.async_copy(dst_hbm.at[off + j + 2], didx.at[n2], semI)

        return 0

    lax.fori_loop(0, EPW_CHUNKS, body, 0)

    plsc.subcore_barrier()

    pltpu.sync_copy(acc_sh.at[pl.ds(row0, RPT)],
                    acc_hbm.at[c, pl.ds(row0, RPT)])


def _linear_body(x_ref, w_ref, deg_ref, y_ref, yb_ref):
    deg = jnp.sum(deg_ref[...], axis=1) + 1.0
    dinv = lax.rsqrt(deg)
    xw = jnp.dot(x_ref[...], w_ref[...], preferred_element_type=jnp.float32)
    y = dinv[:, None] * xw
    y_ref[...] = y
    yb_ref[...] = y


def _combine_body(acc_ref, y_ref, deg_ref, b_ref, o_ref):
    deg = jnp.sum(deg_ref[...], axis=1) + 1.0
    dinv = lax.rsqrt(deg)
    total = acc_ref[0] + acc_ref[1] + y_ref[...]
    o_ref[...] = dinv[:, None] * total + b_ref[...]


_ROWS_BLK = 1000


def kernel(x, edge_index, W, b):
    src = edge_index[0].astype(jnp.int32)
    dst = edge_index[1].astype(jnp.int32)
    pad = E_PAD - E
    src_p = jnp.concatenate([src, jnp.zeros((pad,), jnp.int32)])
    dst_pad = DUMMY + (jnp.arange(pad, dtype=jnp.int32) % (N_PAD - N))
    dst_p = jnp.concatenate([dst, dst_pad])

    deg_part = _deg_kernel(dst_p.reshape(NW, EPW)).T

    y, yb = pl.pallas_call(
        _linear_body,
        grid=(N // _ROWS_BLK,),
        in_specs=[
            pl.BlockSpec((_ROWS_BLK, CH), lambda i: (i, 0)),
            pl.BlockSpec((CH, CH), lambda i: (0, 0)),
            pl.BlockSpec((_ROWS_BLK, NW), lambda i: (i, 0)),
        ],
        out_specs=[
            pl.BlockSpec((_ROWS_BLK, CH), lambda i: (i, 0)),
            pl.BlockSpec((_ROWS_BLK, CH), lambda i: (i, 0)),
        ],
        out_shape=[
            jax.ShapeDtypeStruct((N, CH), jnp.float32),
            jax.ShapeDtypeStruct((N, CH), jnp.float32),
        ],
    )(x, W, deg_part)

    acc = _scatter_kernel(y, yb, src_p.reshape(TOT_CHUNKS, CHUNK),
                          dst_p.reshape(TOT_CHUNKS, CHUNK))

    out = pl.pallas_call(
        _combine_body,
        grid=(N // _ROWS_BLK,),
        in_specs=[
            pl.BlockSpec((NC, _ROWS_BLK, CH), lambda i: (0, i, 0)),
            pl.BlockSpec((_ROWS_BLK, CH), lambda i: (i, 0)),
            pl.BlockSpec((_ROWS_BLK, NW), lambda i: (i, 0)),
            pl.BlockSpec((1, CH), lambda i: (0, 0)),
        ],
        out_specs=pl.BlockSpec((_ROWS_BLK, CH), lambda i: (i, 0)),
        out_shape=jax.ShapeDtypeStruct((N, CH), jnp.float32),
    )(acc, y, deg_part, b.reshape(1, CH))

    return out


# speedup vs baseline: 1.3789x; 1.1638x over previous
import functools

import jax
import jax.numpy as jnp
from jax import lax
from jax.experimental import pallas as pl
from jax.experimental.pallas import tpu as pltpu
from jax.experimental.pallas import tpu_sc as plsc

N = 10000
E = 320000
CH = 128

NC = 2
NS = 16
NW = NC * NS

CHUNK = 128
EPW_CHUNKS = -(-E // (NW * CHUNK))
EPW = EPW_CHUNKS * CHUNK
E_PAD = EPW * NW
TOT_CHUNKS = E_PAD // CHUNK
CPT_A = EPW_CHUNKS
CPT_B = EPW_CHUNKS
N_PAD = 10112
RPT = N_PAD // NS
DUMMY = N

_mesh = plsc.VectorSubcoreMesh(core_axis_name="c", subcore_axis_name="s")
_sc_params = pltpu.CompilerParams(needs_layout_passes=False)


@functools.partial(
    pl.kernel,
    out_type=jax.ShapeDtypeStruct((NW, N_PAD), jnp.float32),
    mesh=_mesh,
    compiler_params=_sc_params,
    scratch_types=[
        pltpu.VMEM((EPW,), jnp.int32),
        pltpu.VMEM((N_PAD,), jnp.float32),
    ],
)
def _deg_kernel(dst_hbm, out_hbm, dstv, degv):
    wid = lax.axis_index("s") * NC + lax.axis_index("c")
    pltpu.sync_copy(dst_hbm.at[wid], dstv)

    zeros16 = jnp.zeros((16,), jnp.float32)
    ones16 = jnp.ones((16,), jnp.float32)

    def zero_body(i, _):
        degv[pl.ds(i * 16, 16)] = zeros16
        return 0

    lax.fori_loop(0, N_PAD // 16, zero_body, 0)

    def hist_body(i, _):
        idx = dstv[pl.ds(i * 16, 16)]
        plsc.addupdate_scatter(degv, [idx], ones16)
        return 0

    lax.fori_loop(0, EPW // 16, hist_body, 0)
    pltpu.sync_copy(degv, out_hbm.at[wid])


@functools.partial(
    pl.kernel,
    out_type=jax.ShapeDtypeStruct((NC, N_PAD, CH), jnp.float32),
    mesh=_mesh,
    compiler_params=_sc_params,
    scratch_types=[
        pltpu.VMEM((4, CHUNK), jnp.int32),
        pltpu.VMEM((4, CHUNK), jnp.int32),
        pltpu.VMEM((3, CHUNK, CH), jnp.float32),
        pltpu.VMEM_SHARED((N_PAD, CH), jnp.float32),
        pltpu.SemaphoreType.DMA,
        pltpu.SemaphoreType.DMA,
        pltpu.SemaphoreType.DMA,
    ],
)
def _scatter_kernel(ya_hbm, yb_hbm, src_hbm, dst_hbm, acc_hbm,
                    sidx, didx, gbuf, acc_sh, semI, semG, semS):
    c = lax.axis_index("c")
    s = lax.axis_index("s")
    off = jnp.where(c == 0, s * CPT_A, NS * CPT_A + s * CPT_B)
    nchunks = jnp.where(c == 0, CPT_A, CPT_B)

    pltpu.sync_copy(src_hbm.at[off], sidx.at[0])
    pltpu.sync_copy(dst_hbm.at[off], didx.at[0])

    pltpu.async_copy(src_hbm.at[off + 1], sidx.at[1], semI)
    pltpu.async_copy(dst_hbm.at[off + 1], didx.at[1], semI)

    @pl.when(c == 0)
    def _():
        pltpu.async_copy(ya_hbm.at[sidx.at[0]], gbuf.at[0], semG)

    @pl.when(c == 1)
    def _():
        pltpu.async_copy(yb_hbm.at[sidx.at[0]], gbuf.at[0], semG)

    zeros16 = jnp.zeros((16,), jnp.float32)

    def zero_body(i, _):
        gbuf[2, i // (CH // 16), pl.ds((i % (CH // 16)) * 16, 16)] = zeros16
        return 0

    lax.fori_loop(0, CHUNK * CH // 16, zero_body, 0)

    row0 = s * RPT
    pltpu.sync_copy(gbuf.at[2], acc_sh.at[pl.ds(row0, CHUNK)])
    pltpu.sync_copy(gbuf.at[2], acc_sh.at[pl.ds(row0 + CHUNK, CHUNK)])
    pltpu.sync_copy(gbuf.at[2], acc_sh.at[pl.ds(row0 + 2 * CHUNK, CHUNK)])
    pltpu.sync_copy(gbuf.at[2], acc_sh.at[pl.ds(row0 + 3 * CHUNK, CHUNK)])
    pltpu.sync_copy(gbuf.at[2, pl.ds(0, RPT - 4 * CHUNK)],
                    acc_sh.at[pl.ds(row0 + 4 * CHUNK, RPT - 4 * CHUNK)])
    plsc.subcore_barrier()

    def _make_body(y_hbm):
        def body(j, _):
            g_cur = j % 3
            g_nxt = (j + 1) % 3
            i_cur = j % 4

            @pl.when(j >= 2)
            def _():
                pltpu.make_async_copy(gbuf.at[g_nxt],
                                      acc_sh.at[pl.ds(row0, CHUNK)],
                                      semS).wait()

            @pl.when(j + 1 < nchunks)
            def _():
                i_nxt = (j + 1) % 4
                pltpu.make_async_copy(src_hbm.at[off + j + 1], sidx.at[i_nxt],
                                      semI).wait()
                pltpu.make_async_copy(dst_hbm.at[off + j + 1], didx.at[i_nxt],
                                      semI).wait()
                pltpu.async_copy(y_hbm.at[sidx.at[i_nxt]], gbuf.at[g_nxt],
                                 semG)

            @pl.when(j + 2 < nchunks)
            def _():
                i_2 = (j + 2) % 4
                pltpu.async_copy(src_hbm.at[off + j + 2], sidx.at[i_2], semI)
                pltpu.async_copy(dst_hbm.at[off + j + 2], didx.at[i_2], semI)

            pltpu.make_async_copy(y_hbm.at[sidx.at[i_cur]], gbuf.at[g_cur],
                                  semG).wait()
            pltpu.async_copy(gbuf.at[g_cur], acc_sh.at[didx.at[i_cur]], semS,
                             add=True)
            return 0

        return body

    @pl.when(c == 0)
    def _():
        lax.fori_loop(0, EPW_CHUNKS, _make_body(ya_hbm), 0)

    @pl.when(c == 1)
    def _():
        lax.fori_loop(0, EPW_CHUNKS, _make_body(yb_hbm), 0)

    pltpu.make_async_copy(gbuf.at[0], acc_sh.at[pl.ds(row0, CHUNK)],
                          semS).wait()
    pltpu.make_async_copy(gbuf.at[0], acc_sh.at[pl.ds(row0, CHUNK)],
                          semS).wait()
    plsc.subcore_barrier()

    pltpu.sync_copy(acc_sh.at[pl.ds(row0, RPT)],
                    acc_hbm.at[c, pl.ds(row0, RPT)])


def _linear_body(x_ref, w_ref, deg_ref, y_ref, yb_ref):
    deg = jnp.sum(deg_ref[...], axis=1) + 1.0
    dinv = lax.rsqrt(deg)
    xw = jnp.dot(x_ref[...], w_ref[...], preferred_element_type=jnp.float32)
    y = dinv[:, None] * xw
    y_ref[...] = y
    yb_ref[...] = y


def _combine_body(acc_ref, y_ref, deg_ref, b_ref, o_ref):
    deg = jnp.sum(deg_ref[...], axis=1) + 1.0
    dinv = lax.rsqrt(deg)
    total = acc_ref[0] + acc_ref[1] + y_ref[...]
    o_ref[...] = dinv[:, None] * total + b_ref[...]


_ROWS_BLK = 1000


def kernel(x, edge_index, W, b):
    src = edge_index[0].astype(jnp.int32)
    dst = edge_index[1].astype(jnp.int32)
    pad = E_PAD - E
    src_p = jnp.concatenate([src, jnp.zeros((pad,), jnp.int32)])
    dst_pad = DUMMY + (jnp.arange(pad, dtype=jnp.int32) % (N_PAD - N))
    dst_p = jnp.concatenate([dst, dst_pad])

    deg_part = _deg_kernel(dst_p.reshape(NW, EPW)).T

    y, yb = pl.pallas_call(
        _linear_body,
        grid=(N // _ROWS_BLK,),
        in_specs=[
            pl.BlockSpec((_ROWS_BLK, CH), lambda i: (i, 0)),
            pl.BlockSpec((CH, CH), lambda i: (0, 0)),
            pl.BlockSpec((_ROWS_BLK, NW), lambda i: (i, 0)),
        ],
        out_specs=[
            pl.BlockSpec((_ROWS_BLK, CH), lambda i: (i, 0)),
            pl.BlockSpec((_ROWS_BLK, CH), lambda i: (i, 0)),
        ],
        out_shape=[
            jax.ShapeDtypeStruct((N, CH), jnp.float32),
            jax.ShapeDtypeStruct((N, CH), jnp.float32),
        ],
    )(x, W, deg_part)

    acc = _scatter_kernel(y, yb, src_p.reshape(TOT_CHUNKS, CHUNK),
                          dst_p.reshape(TOT_CHUNKS, CHUNK))

    out = pl.pallas_call(
        _combine_body,
        grid=(N // _ROWS_BLK,),
        in_specs=[
            pl.BlockSpec((NC, _ROWS_BLK, CH), lambda i: (0, i, 0)),
            pl.BlockSpec((_ROWS_BLK, CH), lambda i: (i, 0)),
            pl.BlockSpec((_ROWS_BLK, NW), lambda i: (i, 0)),
            pl.BlockSpec((1, CH), lambda i: (0, 0)),
        ],
        out_specs=pl.BlockSpec((_ROWS_BLK, CH), lambda i: (i, 0)),
        out_shape=jax.ShapeDtypeStruct((N, CH), jnp.float32),
    )(acc, y, deg_part, b.reshape(1, CH))

    return out


# speedup vs baseline: 1.4530x; 1.0538x over previous
import functools

import jax
import jax.numpy as jnp
from jax import lax
from jax.experimental import pallas as pl
from jax.experimental.pallas import tpu as pltpu
from jax.experimental.pallas import tpu_sc as plsc

N = 10000
E = 320000
CH = 128

NC = 2
NS = 16
NW = NC * NS

CHUNK = 128
EPW_CHUNKS = -(-E // (NW * CHUNK))
EPW = EPW_CHUNKS * CHUNK
E_PAD = EPW * NW
TOT_CHUNKS = E_PAD // CHUNK
CPT_A = 114
CPT_B = 44
N_PAD = 10112
RPT = N_PAD // NS
DUMMY = N

_mesh = plsc.VectorSubcoreMesh(core_axis_name="c", subcore_axis_name="s")
_sc_params = pltpu.CompilerParams(needs_layout_passes=False)


@functools.partial(
    pl.kernel,
    out_type=jax.ShapeDtypeStruct((NW, N_PAD), jnp.float32),
    mesh=_mesh,
    compiler_params=_sc_params,
    scratch_types=[
        pltpu.VMEM((EPW,), jnp.int32),
        pltpu.VMEM((N_PAD,), jnp.float32),
    ],
)
def _deg_kernel(dst_hbm, out_hbm, dstv, degv):
    wid = lax.axis_index("s") * NC + lax.axis_index("c")
    pltpu.sync_copy(dst_hbm.at[wid], dstv)

    zeros16 = jnp.zeros((16,), jnp.float32)
    ones16 = jnp.ones((16,), jnp.float32)

    def zero_body(i, _):
        degv[pl.ds(i * 16, 16)] = zeros16
        return 0

    lax.fori_loop(0, N_PAD // 16, zero_body, 0)

    def hist_body(i, _):
        idx = dstv[pl.ds(i * 16, 16)]
        plsc.addupdate_scatter(degv, [idx], ones16)
        return 0

    lax.fori_loop(0, EPW // 16, hist_body, 0)
    pltpu.sync_copy(degv, out_hbm.at[wid])


@functools.partial(
    pl.kernel,
    out_type=jax.ShapeDtypeStruct((NC, N_PAD, CH), jnp.float32),
    mesh=_mesh,
    compiler_params=_sc_params,
    scratch_types=[
        pltpu.VMEM((4, CHUNK), jnp.int32),
        pltpu.VMEM((4, CHUNK), jnp.int32),
        pltpu.VMEM((3, CHUNK, CH), jnp.float32),
        pltpu.VMEM_SHARED((N_PAD, CH), jnp.float32),
        pltpu.SemaphoreType.DMA,
        pltpu.SemaphoreType.DMA,
        pltpu.SemaphoreType.DMA,
    ],
)
def _scatter_kernel(ya_hbm, yb_hbm, src_hbm, dst_hbm, acc_hbm,
                    sidx, didx, gbuf, acc_sh, semI, semG, semS):
    c = lax.axis_index("c")
    s = lax.axis_index("s")
    off = jnp.where(c == 0, s * CPT_A, NS * CPT_A + s * CPT_B)
    nchunks = jnp.where(c == 0, CPT_A, CPT_B)

    pltpu.sync_copy(src_hbm.at[off], sidx.at[0])
    pltpu.sync_copy(dst_hbm.at[off], didx.at[0])

    pltpu.async_copy(src_hbm.at[off + 1], sidx.at[1], semI)
    pltpu.async_copy(dst_hbm.at[off + 1], didx.at[1], semI)

    @pl.when(c == 0)
    def _():
        pltpu.async_copy(ya_hbm.at[sidx.at[0]], gbuf.at[0], semG)

    @pl.when(c == 1)
    def _():
        pltpu.async_copy(yb_hbm.at[sidx.at[0]], gbuf.at[0], semG)

    zeros16 = jnp.zeros((16,), jnp.float32)

    def zero_body(i, _):
        gbuf[2, i // (CH // 16), pl.ds((i % (CH // 16)) * 16, 16)] = zeros16
        return 0

    lax.fori_loop(0, CHUNK * CH // 16, zero_body, 0)

    row0 = s * RPT
    pltpu.sync_copy(gbuf.at[2], acc_sh.at[pl.ds(row0, CHUNK)])
    pltpu.sync_copy(gbuf.at[2], acc_sh.at[pl.ds(row0 + CHUNK, CHUNK)])
    pltpu.sync_copy(gbuf.at[2], acc_sh.at[pl.ds(row0 + 2 * CHUNK, CHUNK)])
    pltpu.sync_copy(gbuf.at[2], acc_sh.at[pl.ds(row0 + 3 * CHUNK, CHUNK)])
    pltpu.sync_copy(gbuf.at[2, pl.ds(0, RPT - 4 * CHUNK)],
                    acc_sh.at[pl.ds(row0 + 4 * CHUNK, RPT - 4 * CHUNK)])
    plsc.subcore_barrier()

    def _make_body(y_hbm):
        def body(j, _):
            g_cur = j % 3
            g_nxt = (j + 1) % 3
            i_cur = j % 4

            @pl.when(j >= 2)
            def _():
                pltpu.make_async_copy(gbuf.at[g_nxt],
                                      acc_sh.at[pl.ds(row0, CHUNK)],
                                      semS).wait()

            @pl.when(j + 1 < nchunks)
            def _():
                i_nxt = (j + 1) % 4
                pltpu.make_async_copy(src_hbm.at[off + j + 1], sidx.at[i_nxt],
                                      semI).wait()
                pltpu.make_async_copy(dst_hbm.at[off + j + 1], didx.at[i_nxt],
                                      semI).wait()
                pltpu.async_copy(y_hbm.at[sidx.at[i_nxt]], gbuf.at[g_nxt],
                                 semG)

            @pl.when(j + 2 < nchunks)
            def _():
                i_2 = (j + 2) % 4
                pltpu.async_copy(src_hbm.at[off + j + 2], sidx.at[i_2], semI)
                pltpu.async_copy(dst_hbm.at[off + j + 2], didx.at[i_2], semI)

            pltpu.make_async_copy(y_hbm.at[sidx.at[i_cur]], gbuf.at[g_cur],
                                  semG).wait()
            pltpu.async_copy(gbuf.at[g_cur], acc_sh.at[didx.at[i_cur]], semS,
                             add=True)
            return 0

        return body

    @pl.when(c == 0)
    def _():
        lax.fori_loop(0, CPT_A, _make_body(ya_hbm), 0)

    @pl.when(c == 1)
    def _():
        lax.fori_loop(0, CPT_B, _make_body(yb_hbm), 0)

    pltpu.make_async_copy(gbuf.at[0], acc_sh.at[pl.ds(row0, CHUNK)],
                          semS).wait()
    pltpu.make_async_copy(gbuf.at[0], acc_sh.at[pl.ds(row0, CHUNK)],
                          semS).wait()
    plsc.subcore_barrier()

    pltpu.sync_copy(acc_sh.at[pl.ds(row0, RPT)],
                    acc_hbm.at[c, pl.ds(row0, RPT)])


def _linear_body(x_ref, w_ref, deg_ref, y_ref, yb_ref):
    deg = jnp.sum(deg_ref[...], axis=1) + 1.0
    dinv = lax.rsqrt(deg)
    xw = jnp.dot(x_ref[...], w_ref[...], preferred_element_type=jnp.float32)
    y = dinv[:, None] * xw
    y_ref[...] = y
    yb_ref[...] = y


def _combine_body(acc_ref, y_ref, deg_ref, b_ref, o_ref):
    deg = jnp.sum(deg_ref[...], axis=1) + 1.0
    dinv = lax.rsqrt(deg)
    total = acc_ref[0] + acc_ref[1] + y_ref[...]
    o_ref[...] = dinv[:, None] * total + b_ref[...]


_ROWS_BLK = 1000


def kernel(x, edge_index, W, b):
    src = edge_index[0].astype(jnp.int32)
    dst = edge_index[1].astype(jnp.int32)
    pad = E_PAD - E
    src_p = jnp.concatenate([src, jnp.zeros((pad,), jnp.int32)])
    dst_pad = DUMMY + (jnp.arange(pad, dtype=jnp.int32) % (N_PAD - N))
    dst_p = jnp.concatenate([dst, dst_pad])

    deg_part = _deg_kernel(dst_p.reshape(NW, EPW)).T

    y, yb = pl.pallas_call(
        _linear_body,
        grid=(N // _ROWS_BLK,),
        in_specs=[
            pl.BlockSpec((_ROWS_BLK, CH), lambda i: (i, 0)),
            pl.BlockSpec((CH, CH), lambda i: (0, 0)),
            pl.BlockSpec((_ROWS_BLK, NW), lambda i: (i, 0)),
        ],
        out_specs=[
            pl.BlockSpec((_ROWS_BLK, CH), lambda i: (i, 0)),
            pl.BlockSpec((_ROWS_BLK, CH), lambda i: (i, 0)),
        ],
        out_shape=[
            jax.ShapeDtypeStruct((N, CH), jnp.float32),
            jax.ShapeDtypeStruct((N, CH), jnp.float32),
        ],
    )(x, W, deg_part)

    acc = _scatter_kernel(y, yb, src_p.reshape(TOT_CHUNKS, CHUNK),
                          dst_p.reshape(TOT_CHUNKS, CHUNK))

    out = pl.pallas_call(
        _combine_body,
        grid=(N // _ROWS_BLK,),
        in_specs=[
            pl.BlockSpec((NC, _ROWS_BLK, CH), lambda i: (0, i, 0)),
            pl.BlockSpec((_ROWS_BLK, CH), lambda i: (i, 0)),
            pl.BlockSpec((_ROWS_BLK, NW), lambda i: (i, 0)),
            pl.BlockSpec((1, CH), lambda i: (0, 0)),
        ],
        out_specs=pl.BlockSpec((_ROWS_BLK, CH), lambda i: (i, 0)),
        out_shape=jax.ShapeDtypeStruct((N, CH), jnp.float32),
    )(acc, y, deg_part, b.reshape(1, CH))

    return out


# speedup vs baseline: 1.5229x; 1.0481x over previous
import functools

import jax
import jax.numpy as jnp
from jax import lax
from jax.experimental import pallas as pl
from jax.experimental.pallas import tpu as pltpu
from jax.experimental.pallas import tpu_sc as plsc

N = 10000
E = 320000
CH = 128

NC = 2
NS = 16
NW = NC * NS

CHUNK = 128
EPW_CHUNKS = -(-E // (NW * CHUNK))
EPW = EPW_CHUNKS * CHUNK
E_PAD = EPW * NW
TOT_CHUNKS = E_PAD // CHUNK
CPT_A = 140
CPT_B = 18
N_PAD = 10112
RPT = N_PAD // NS
DUMMY = N

_mesh = plsc.VectorSubcoreMesh(core_axis_name="c", subcore_axis_name="s")
_sc_params = pltpu.CompilerParams(needs_layout_passes=False)


@functools.partial(
    pl.kernel,
    out_type=jax.ShapeDtypeStruct((NW, N_PAD), jnp.float32),
    mesh=_mesh,
    compiler_params=_sc_params,
    scratch_types=[
        pltpu.VMEM((EPW,), jnp.int32),
        pltpu.VMEM((N_PAD,), jnp.float32),
    ],
)
def _deg_kernel(dst_hbm, out_hbm, dstv, degv):
    wid = lax.axis_index("s") * NC + lax.axis_index("c")
    pltpu.sync_copy(dst_hbm.at[wid], dstv)

    zeros16 = jnp.zeros((16,), jnp.float32)
    ones16 = jnp.ones((16,), jnp.float32)

    def zero_body(i, _):
        degv[pl.ds(i * 16, 16)] = zeros16
        return 0

    lax.fori_loop(0, N_PAD // 16, zero_body, 0)

    def hist_body(i, _):
        idx = dstv[pl.ds(i * 16, 16)]
        plsc.addupdate_scatter(degv, [idx], ones16)
        return 0

    lax.fori_loop(0, EPW // 16, hist_body, 0)
    pltpu.sync_copy(degv, out_hbm.at[wid])


@functools.partial(
    pl.kernel,
    out_type=jax.ShapeDtypeStruct((NC, N_PAD, CH), jnp.float32),
    mesh=_mesh,
    compiler_params=_sc_params,
    scratch_types=[
        pltpu.VMEM((4, CHUNK), jnp.int32),
        pltpu.VMEM((4, CHUNK), jnp.int32),
        pltpu.VMEM((3, CHUNK, CH), jnp.float32),
        pltpu.VMEM_SHARED((N_PAD, CH), jnp.float32),
        pltpu.SemaphoreType.DMA,
        pltpu.SemaphoreType.DMA,
        pltpu.SemaphoreType.DMA,
    ],
)
def _scatter_kernel(ya_hbm, yb_hbm, src_hbm, dst_hbm, acc_hbm,
                    sidx, didx, gbuf, acc_sh, semI, semG, semS):
    c = lax.axis_index("c")
    s = lax.axis_index("s")
    off = jnp.where(c == 0, s * CPT_A, NS * CPT_A + s * CPT_B)
    nchunks = jnp.where(c == 0, CPT_A, CPT_B)

    pltpu.sync_copy(src_hbm.at[off], sidx.at[0])
    pltpu.sync_copy(dst_hbm.at[off], didx.at[0])

    pltpu.async_copy(src_hbm.at[off + 1], sidx.at[1], semI)
    pltpu.async_copy(dst_hbm.at[off + 1], didx.at[1], semI)

    @pl.when(c == 0)
    def _():
        pltpu.async_copy(ya_hbm.at[sidx.at[0]], gbuf.at[0], semG)

    @pl.when(c == 1)
    def _():
        pltpu.async_copy(yb_hbm.at[sidx.at[0]], gbuf.at[0], semG)

    zeros16 = jnp.zeros((16,), jnp.float32)

    def zero_body(i, _):
        gbuf[2, i // (CH // 16), pl.ds((i % (CH // 16)) * 16, 16)] = zeros16
        return 0

    lax.fori_loop(0, CHUNK * CH // 16, zero_body, 0)

    row0 = s * RPT
    pltpu.sync_copy(gbuf.at[2], acc_sh.at[pl.ds(row0, CHUNK)])
    pltpu.sync_copy(gbuf.at[2], acc_sh.at[pl.ds(row0 + CHUNK, CHUNK)])
    pltpu.sync_copy(gbuf.at[2], acc_sh.at[pl.ds(row0 + 2 * CHUNK, CHUNK)])
    pltpu.sync_copy(gbuf.at[2], acc_sh.at[pl.ds(row0 + 3 * CHUNK, CHUNK)])
    pltpu.sync_copy(gbuf.at[2, pl.ds(0, RPT - 4 * CHUNK)],
                    acc_sh.at[pl.ds(row0 + 4 * CHUNK, RPT - 4 * CHUNK)])
    plsc.subcore_barrier()

    def _make_body(y_hbm):
        def body(j, _):
            g_cur = j % 3
            g_nxt = (j + 1) % 3
            i_cur = j % 4

            @pl.when(j >= 2)
            def _():
                pltpu.make_async_copy(gbuf.at[g_nxt],
                                      acc_sh.at[pl.ds(row0, CHUNK)],
                                      semS).wait()

            @pl.when(j + 1 < nchunks)
            def _():
                i_nxt = (j + 1) % 4
                pltpu.make_async_copy(src_hbm.at[off + j + 1], sidx.at[i_nxt],
                                      semI).wait()
                pltpu.make_async_copy(dst_hbm.at[off + j + 1], didx.at[i_nxt],
                                      semI).wait()
                pltpu.async_copy(y_hbm.at[sidx.at[i_nxt]], gbuf.at[g_nxt],
                                 semG)

            @pl.when(j + 2 < nchunks)
            def _():
                i_2 = (j + 2) % 4
                pltpu.async_copy(src_hbm.at[off + j + 2], sidx.at[i_2], semI)
                pltpu.async_copy(dst_hbm.at[off + j + 2], didx.at[i_2], semI)

            pltpu.make_async_copy(y_hbm.at[sidx.at[i_cur]], gbuf.at[g_cur],
                                  semG).wait()
            pltpu.async_copy(gbuf.at[g_cur], acc_sh.at[didx.at[i_cur]], semS,
                             add=True)
            return 0

        return body

    @pl.when(c == 0)
    def _():
        lax.fori_loop(0, CPT_A, _make_body(ya_hbm), 0)

    @pl.when(c == 1)
    def _():
        lax.fori_loop(0, CPT_B, _make_body(yb_hbm), 0)

    pltpu.make_async_copy(gbuf.at[0], acc_sh.at[pl.ds(row0, CHUNK)],
                          semS).wait()
    pltpu.make_async_copy(gbuf.at[0], acc_sh.at[pl.ds(row0, CHUNK)],
                          semS).wait()
    plsc.subcore_barrier()

    pltpu.sync_copy(acc_sh.at[pl.ds(row0, RPT)],
                    acc_hbm.at[c, pl.ds(row0, RPT)])


def _linear_body(x_ref, w_ref, deg_ref, y_ref, yb_ref):
    deg = jnp.sum(deg_ref[...], axis=1) + 1.0
    dinv = lax.rsqrt(deg)
    xw = jnp.dot(x_ref[...], w_ref[...], preferred_element_type=jnp.float32)
    y = dinv[:, None] * xw
    y_ref[...] = y
    yb_ref[...] = y


def _combine_body(acc_ref, y_ref, deg_ref, b_ref, o_ref):
    deg = jnp.sum(deg_ref[...], axis=1) + 1.0
    dinv = lax.rsqrt(deg)
    total = acc_ref[0] + acc_ref[1] + y_ref[...]
    o_ref[...] = dinv[:, None] * total + b_ref[...]


_ROWS_BLK = 1000


def kernel(x, edge_index, W, b):
    src = edge_index[0].astype(jnp.int32)
    dst = edge_index[1].astype(jnp.int32)
    pad = E_PAD - E
    src_p = jnp.concatenate([src, jnp.zeros((pad,), jnp.int32)])
    dst_pad = DUMMY + (jnp.arange(pad, dtype=jnp.int32) % (N_PAD - N))
    dst_p = jnp.concatenate([dst, dst_pad])

    deg_part = _deg_kernel(dst_p.reshape(NW, EPW)).T

    y, yb = pl.pallas_call(
        _linear_body,
        grid=(N // _ROWS_BLK,),
        in_specs=[
            pl.BlockSpec((_ROWS_BLK, CH), lambda i: (i, 0)),
            pl.BlockSpec((CH, CH), lambda i: (0, 0)),
            pl.BlockSpec((_ROWS_BLK, NW), lambda i: (i, 0)),
        ],
        out_specs=[
            pl.BlockSpec((_ROWS_BLK, CH), lambda i: (i, 0)),
            pl.BlockSpec((_ROWS_BLK, CH), lambda i: (i, 0)),
        ],
        out_shape=[
            jax.ShapeDtypeStruct((N, CH), jnp.float32),
            jax.ShapeDtypeStruct((N, CH), jnp.float32),
        ],
    )(x, W, deg_part)

    acc = _scatter_kernel(y, yb, src_p.reshape(TOT_CHUNKS, CHUNK),
                          dst_p.reshape(TOT_CHUNKS, CHUNK))

    out = pl.pallas_call(
        _combine_body,
        grid=(N // _ROWS_BLK,),
        in_specs=[
            pl.BlockSpec((NC, _ROWS_BLK, CH), lambda i: (0, i, 0)),
            pl.BlockSpec((_ROWS_BLK, CH), lambda i: (i, 0)),
            pl.BlockSpec((_ROWS_BLK, NW), lambda i: (i, 0)),
            pl.BlockSpec((1, CH), lambda i: (0, 0)),
        ],
        out_specs=pl.BlockSpec((_ROWS_BLK, CH), lambda i: (i, 0)),
        out_shape=jax.ShapeDtypeStruct((N, CH), jnp.float32),
    )(acc, y, deg_part, b.reshape(1, CH))

    return out


# speedup vs baseline: 1.8745x; 1.2309x over previous
import functools

import jax
import jax.numpy as jnp
from jax import lax
from jax.experimental import pallas as pl
from jax.experimental.pallas import tpu as pltpu
from jax.experimental.pallas import tpu_sc as plsc

N = 10000
E = 320000
CH = 128

NC = 2
NS = 16
NW = NC * NS

CHUNK = 128
TOT_CHUNKS = E // CHUNK
N_PAD = 10112
RPT = N_PAD // NS

CPT_A = 152
CPT_REM = 4
CPT_B = 4
_C1_BASE = NS * CPT_A + CPT_REM

EPW = E // NW

_mesh = plsc.VectorSubcoreMesh(core_axis_name="c", subcore_axis_name="s")
_sc_params = pltpu.CompilerParams(needs_layout_passes=False)


@functools.partial(
    pl.kernel,
    out_type=jax.ShapeDtypeStruct((NW, N_PAD), jnp.float32),
    mesh=_mesh,
    compiler_params=_sc_params,
    scratch_types=[
        pltpu.VMEM((EPW,), jnp.int32),
        pltpu.VMEM((N_PAD,), jnp.float32),
    ],
)
def _deg_kernel(dst_hbm, out_hbm, dstv, degv):
    wid = lax.axis_index("s") * NC + lax.axis_index("c")
    pltpu.sync_copy(dst_hbm.at[wid], dstv)

    zeros16 = jnp.zeros((16,), jnp.float32)
    ones16 = jnp.ones((16,), jnp.float32)

    def zero_body(i, _):
        degv[pl.ds(i * 16, 16)] = zeros16
        return 0

    lax.fori_loop(0, N_PAD // 16, zero_body, 0)

    def hist_body(i, _):
        idx = dstv[pl.ds(i * 16, 16)]
        plsc.addupdate_scatter(degv, [idx], ones16)
        return 0

    lax.fori_loop(0, EPW // 16, hist_body, 0)
    pltpu.sync_copy(degv, out_hbm.at[wid])


@functools.partial(
    pl.kernel,
    out_type=jax.ShapeDtypeStruct((NC, N_PAD, CH), jnp.float32),
    mesh=_mesh,
    compiler_params=_sc_params,
    scratch_types=[
        pltpu.VMEM((4, CHUNK), jnp.int32),
        pltpu.VMEM((4, CHUNK), jnp.int32),
        pltpu.VMEM((3, CHUNK, CH), jnp.float32),
        pltpu.VMEM_SHARED((N_PAD, CH), jnp.float32),
        pltpu.SemaphoreType.DMA,
        pltpu.SemaphoreType.DMA,
        pltpu.SemaphoreType.DMA,
    ],
)
def _scatter_kernel(y_hbm, src_hbm, dst_hbm, acc_hbm,
                    sidx, didx, gbuf, acc_sh, semI, semG, semS):
    c = lax.axis_index("c")
    s = lax.axis_index("s")
    off = jnp.where(c == 0, s * CPT_A + jnp.minimum(s, CPT_REM),
                    _C1_BASE + s * CPT_B)
    nchunks = jnp.where(c == 0, CPT_A + (s < CPT_REM).astype(jnp.int32),
                        CPT_B)

    pltpu.sync_copy(src_hbm.at[off], sidx.at[0])
    pltpu.sync_copy(dst_hbm.at[off], didx.at[0])
    pltpu.async_copy(y_hbm.at[sidx.at[0]], gbuf.at[0], semG)
    pltpu.async_copy(src_hbm.at[off + 1], sidx.at[1], semI)
    pltpu.async_copy(dst_hbm.at[off + 1], didx.at[1], semI)

    zeros16 = jnp.zeros((16,), jnp.float32)

    def zero_body(i, _):
        gbuf[2, i // (CH // 16), pl.ds((i % (CH // 16)) * 16, 16)] = zeros16
        return 0

    lax.fori_loop(0, CHUNK * CH // 16, zero_body, 0)

    row0 = s * RPT
    pltpu.sync_copy(gbuf.at[2], acc_sh.at[pl.ds(row0, CHUNK)])
    pltpu.sync_copy(gbuf.at[2], acc_sh.at[pl.ds(row0 + CHUNK, CHUNK)])
    pltpu.sync_copy(gbuf.at[2], acc_sh.at[pl.ds(row0 + 2 * CHUNK, CHUNK)])
    pltpu.sync_copy(gbuf.at[2], acc_sh.at[pl.ds(row0 + 3 * CHUNK, CHUNK)])
    pltpu.sync_copy(gbuf.at[2, pl.ds(0, RPT - 4 * CHUNK)],
                    acc_sh.at[pl.ds(row0 + 4 * CHUNK, RPT - 4 * CHUNK)])
    plsc.subcore_barrier()

    def body(j, _):
        g_cur = j % 3
        g_nxt = (j + 1) % 3
        i_cur = j % 4

        @pl.when(j >= 2)
        def _():
            pltpu.make_async_copy(gbuf.at[g_nxt],
                                  acc_sh.at[pl.ds(row0, CHUNK)],
                                  semS).wait()

        @pl.when(j + 1 < nchunks)
        def _():
            i_nxt = (j + 1) % 4
            pltpu.make_async_copy(src_hbm.at[off + j + 1], sidx.at[i_nxt],
                                  semI).wait()
            pltpu.make_async_copy(dst_hbm.at[off + j + 1], didx.at[i_nxt],
                                  semI).wait()
            pltpu.async_copy(y_hbm.at[sidx.at[i_nxt]], gbuf.at[g_nxt], semG)

        @pl.when(j + 2 < nchunks)
        def _():
            i_2 = (j + 2) % 4
            pltpu.async_copy(src_hbm.at[off + j + 2], sidx.at[i_2], semI)
            pltpu.async_copy(dst_hbm.at[off + j + 2], didx.at[i_2], semI)

        pltpu.make_async_copy(y_hbm.at[sidx.at[i_cur]], gbuf.at[g_cur],
                              semG).wait()
        pltpu.async_copy(gbuf.at[g_cur], acc_sh.at[didx.at[i_cur]], semS,
                         add=True)
        return 0

    lax.fori_loop(0, nchunks, body, 0)

    pltpu.make_async_copy(gbuf.at[0], acc_sh.at[pl.ds(row0, CHUNK)],
                          semS).wait()
    pltpu.make_async_copy(gbuf.at[0], acc_sh.at[pl.ds(row0, CHUNK)],
                          semS).wait()
    plsc.subcore_barrier()

    pltpu.sync_copy(acc_sh.at[pl.ds(row0, RPT)],
                    acc_hbm.at[c, pl.ds(row0, RPT)])


def _linear_body(x_ref, w_ref, deg_ref, y_ref):
    deg = jnp.sum(deg_ref[...], axis=1) + 1.0
    dinv = lax.rsqrt(deg)
    xw = jnp.dot(x_ref[...], w_ref[...], preferred_element_type=jnp.float32)
    y_ref[...] = dinv[:, None] * xw


def _combine_body(acc_ref, y_ref, deg_ref, b_ref, o_ref):
    deg = jnp.sum(deg_ref[...], axis=1) + 1.0
    dinv = lax.rsqrt(deg)
    total = acc_ref[0] + acc_ref[1] + y_ref[...]
    o_ref[...] = dinv[:, None] * total + b_ref[...]


_ROWS_BLK = 1000


def kernel(x, edge_index, W, b):
    src = edge_index[0].astype(jnp.int32)
    dst = edge_index[1].astype(jnp.int32)

    deg_part = _deg_kernel(dst.reshape(NW, EPW)).T

    y = pl.pallas_call(
        _linear_body,
        grid=(N // _ROWS_BLK,),
        in_specs=[
            pl.BlockSpec((_ROWS_BLK, CH), lambda i: (i, 0)),
            pl.BlockSpec((CH, CH), lambda i: (0, 0)),
            pl.BlockSpec((_ROWS_BLK, NW), lambda i: (i, 0)),
        ],
        out_specs=pl.BlockSpec((_ROWS_BLK, CH), lambda i: (i, 0)),
        out_shape=jax.ShapeDtypeStruct((N, CH), jnp.float32),
    )(x, W, deg_part)

    acc = _scatter_kernel(y, src.reshape(TOT_CHUNKS, CHUNK),
                          dst.reshape(TOT_CHUNKS, CHUNK))

    out = pl.pallas_call(
        _combine_body,
        grid=(N // _ROWS_BLK,),
        in_specs=[
            pl.BlockSpec((NC, _ROWS_BLK, CH), lambda i: (0, i, 0)),
            pl.BlockSpec((_ROWS_BLK, CH), lambda i: (i, 0)),
            pl.BlockSpec((_ROWS_BLK, NW), lambda i: (i, 0)),
            pl.BlockSpec((1, CH), lambda i: (0, 0)),
        ],
        out_specs=pl.BlockSpec((_ROWS_BLK, CH), lambda i: (i, 0)),
        out_shape=jax.ShapeDtypeStruct((N, CH), jnp.float32),
    )(acc, y, deg_part, b.reshape(1, CH))

    return out


# speedup vs baseline: 2.1215x; 1.1318x over previous
import functools

import jax
import jax.numpy as jnp
from jax import lax
from jax.experimental import pallas as pl
from jax.experimental.pallas import tpu as pltpu
from jax.experimental.pallas import tpu_sc as plsc

N = 10000
E = 320000
CH = 128

NC = 2
NS = 16
NW = NC * NS

CHUNK = 128
TOT_CHUNKS = E // CHUNK
N_PAD = 10112
RPT = N_PAD // NS

CPT_A = 146
CPT_REM = 4
CPT_B = 10
_C1_BASE = NS * CPT_A + CPT_REM

DEG_CPT = TOT_CHUNKS // NW
DEG_REM = TOT_CHUNKS % NW
DEG_MAX = (DEG_CPT + 1) * CHUNK

_mesh = plsc.VectorSubcoreMesh(core_axis_name="c", subcore_axis_name="s")
_sc_params = pltpu.CompilerParams(needs_layout_passes=False)


@functools.partial(
    pl.kernel,
    out_type=jax.ShapeDtypeStruct((NW, N_PAD), jnp.float32),
    mesh=_mesh,
    compiler_params=_sc_params,
    scratch_types=[
        pltpu.VMEM((DEG_MAX,), jnp.int32),
        pltpu.VMEM((N_PAD,), jnp.float32),
    ],
)
def _deg_kernel(ei_hbm, out_hbm, dstv, degv):
    wid = lax.axis_index("s") * NC + lax.axis_index("c")
    dgo = wid * DEG_CPT + jnp.minimum(wid, DEG_REM)
    dgn = DEG_CPT + (wid < DEG_REM).astype(jnp.int32)
    pltpu.sync_copy(ei_hbm.at[1, pl.ds(dgo * CHUNK, DEG_CPT * CHUNK)],
                    dstv.at[pl.ds(0, DEG_CPT * CHUNK)])

    @pl.when(wid < DEG_REM)
    def _():
        pltpu.sync_copy(ei_hbm.at[1, pl.ds((dgo + DEG_CPT) * CHUNK, CHUNK)],
                        dstv.at[pl.ds(DEG_CPT * CHUNK, CHUNK)])

    zeros16 = jnp.zeros((16,), jnp.float32)
    ones16 = jnp.ones((16,), jnp.float32)

    def zero_body(i, _):
        degv[pl.ds(i * 16, 16)] = zeros16
        return 0

    lax.fori_loop(0, N_PAD // 16, zero_body, 0)

    def hist_body(i, _):
        idx = dstv[pl.ds(i * 16, 16)]
        plsc.addupdate_scatter(degv, [idx], ones16)
        return 0

    lax.fori_loop(0, dgn * (CHUNK // 16), hist_body, 0)
    pltpu.sync_copy(degv, out_hbm.at[wid])


@functools.partial(
    pl.kernel,
    out_type=jax.ShapeDtypeStruct((NC, N_PAD, CH), jnp.float32),
    mesh=_mesh,
    compiler_params=_sc_params,
    scratch_types=[
        pltpu.VMEM((4, CHUNK), jnp.int32),
        pltpu.VMEM((4, CHUNK), jnp.int32),
        pltpu.VMEM((3, CHUNK, CH), jnp.float32),
        pltpu.VMEM_SHARED((N_PAD, CH), jnp.float32),
        pltpu.SemaphoreType.DMA,
        pltpu.SemaphoreType.DMA,
        pltpu.SemaphoreType.DMA,
    ],
)
def _scatter_kernel(y_hbm, ei_hbm, acc_hbm,
                    sidx, didx, gbuf, acc_sh, semI, semG, semS):
    c = lax.axis_index("c")
    s = lax.axis_index("s")
    off = jnp.where(c == 0, s * CPT_A + jnp.minimum(s, CPT_REM),
                    _C1_BASE + s * CPT_B)
    nchunks = jnp.where(c == 0, CPT_A + (s < CPT_REM).astype(jnp.int32),
                        CPT_B)

    pltpu.sync_copy(ei_hbm.at[0, pl.ds(off * CHUNK, CHUNK)], sidx.at[0])
    pltpu.sync_copy(ei_hbm.at[1, pl.ds(off * CHUNK, CHUNK)], didx.at[0])
    pltpu.async_copy(y_hbm.at[sidx.at[0]], gbuf.at[0], semG)
    pltpu.async_copy(ei_hbm.at[0, pl.ds((off + 1) * CHUNK, CHUNK)],
                     sidx.at[1], semI)
    pltpu.async_copy(ei_hbm.at[1, pl.ds((off + 1) * CHUNK, CHUNK)],
                     didx.at[1], semI)

    zeros16 = jnp.zeros((16,), jnp.float32)

    def zero_body(i, _):
        gbuf[2, i // (CH // 16), pl.ds((i % (CH // 16)) * 16, 16)] = zeros16
        return 0

    lax.fori_loop(0, CHUNK * CH // 16, zero_body, 0)

    row0 = s * RPT
    pltpu.sync_copy(gbuf.at[2], acc_sh.at[pl.ds(row0, CHUNK)])
    pltpu.sync_copy(gbuf.at[2], acc_sh.at[pl.ds(row0 + CHUNK, CHUNK)])
    pltpu.sync_copy(gbuf.at[2], acc_sh.at[pl.ds(row0 + 2 * CHUNK, CHUNK)])
    pltpu.sync_copy(gbuf.at[2], acc_sh.at[pl.ds(row0 + 3 * CHUNK, CHUNK)])
    pltpu.sync_copy(gbuf.at[2, pl.ds(0, RPT - 4 * CHUNK)],
                    acc_sh.at[pl.ds(row0 + 4 * CHUNK, RPT - 4 * CHUNK)])
    plsc.subcore_barrier()

    def body(j, _):
        g_cur = j % 3
        g_nxt = (j + 1) % 3
        i_cur = j % 4

        @pl.when(j >= 2)
        def _():
            pltpu.make_async_copy(gbuf.at[g_nxt],
                                  acc_sh.at[pl.ds(row0, CHUNK)],
                                  semS).wait()

        @pl.when(j + 1 < nchunks)
        def _():
            i_nxt = (j + 1) % 4
            pltpu.make_async_copy(
                ei_hbm.at[0, pl.ds((off + j + 1) * CHUNK, CHUNK)],
                sidx.at[i_nxt], semI).wait()
            pltpu.make_async_copy(
                ei_hbm.at[1, pl.ds((off + j + 1) * CHUNK, CHUNK)],
                didx.at[i_nxt], semI).wait()
            pltpu.async_copy(y_hbm.at[sidx.at[i_nxt]], gbuf.at[g_nxt], semG)

        @pl.when(j + 2 < nchunks)
        def _():
            i_2 = (j + 2) % 4
            pltpu.async_copy(ei_hbm.at[0, pl.ds((off + j + 2) * CHUNK, CHUNK)],
                             sidx.at[i_2], semI)
            pltpu.async_copy(ei_hbm.at[1, pl.ds((off + j + 2) * CHUNK, CHUNK)],
                             didx.at[i_2], semI)

        pltpu.make_async_copy(y_hbm.at[sidx.at[i_cur]], gbuf.at[g_cur],
                              semG).wait()
        pltpu.async_copy(gbuf.at[g_cur], acc_sh.at[didx.at[i_cur]], semS,
                         add=True)
        return 0

    lax.fori_loop(0, nchunks, body, 0)

    pltpu.make_async_copy(gbuf.at[0], acc_sh.at[pl.ds(row0, CHUNK)],
                          semS).wait()
    pltpu.make_async_copy(gbuf.at[0], acc_sh.at[pl.ds(row0, CHUNK)],
                          semS).wait()
    plsc.subcore_barrier()

    pltpu.sync_copy(acc_sh.at[pl.ds(row0, RPT)],
                    acc_hbm.at[c, pl.ds(row0, RPT)])


def _linear_body(x_ref, w_ref, deg_ref, y_ref):
    deg = jnp.sum(deg_ref[...], axis=0) + 1.0
    dinv = lax.rsqrt(deg)
    xw = jnp.dot(x_ref[...], w_ref[...], preferred_element_type=jnp.float32)
    y_ref[...] = dinv[:, None] * xw


def _combine_body(acc_ref, y_ref, deg_ref, b_ref, o_ref):
    deg = jnp.sum(deg_ref[...], axis=0) + 1.0
    dinv = lax.rsqrt(deg)
    total = acc_ref[0] + acc_ref[1] + y_ref[...]
    o_ref[...] = dinv[:, None] * total + b_ref[...]


_ROWS_BLK = 1024


def kernel(x, edge_index, W, b):
    ei = edge_index.astype(jnp.int32)

    deg_part = _deg_kernel(ei)

    grid = (pl.cdiv(N, _ROWS_BLK),)
    y = pl.pallas_call(
        _linear_body,
        grid=grid,
        in_specs=[
            pl.BlockSpec((_ROWS_BLK, CH), lambda i: (i, 0)),
            pl.BlockSpec((CH, CH), lambda i: (0, 0)),
            pl.BlockSpec((NW, _ROWS_BLK), lambda i: (0, i)),
        ],
        out_specs=pl.BlockSpec((_ROWS_BLK, CH), lambda i: (i, 0)),
        out_shape=jax.ShapeDtypeStruct((N, CH), jnp.float32),
    )(x, W, deg_part)

    acc = _scatter_kernel(y, ei)

    out = pl.pallas_call(
        _combine_body,
        grid=grid,
        in_specs=[
            pl.BlockSpec((NC, _ROWS_BLK, CH), lambda i: (0, i, 0)),
            pl.BlockSpec((_ROWS_BLK, CH), lambda i: (i, 0)),
            pl.BlockSpec((NW, _ROWS_BLK), lambda i: (0, i)),
            pl.BlockSpec((1, CH), lambda i: (0, 0)),
        ],
        out_specs=pl.BlockSpec((_ROWS_BLK, CH), lambda i: (i, 0)),
        out_shape=jax.ShapeDtypeStruct((N, CH), jnp.float32),
    )(acc, y, deg_part, b.reshape(1, CH))

    return out


# speedup vs baseline: 2.1743x; 1.0249x over previous
import functools

import jax
import jax.numpy as jnp
from jax import lax
from jax.experimental import pallas as pl
from jax.experimental.pallas import tpu as pltpu
from jax.experimental.pallas import tpu_sc as plsc

N = 10000
E = 320000
CH = 128

NC = 2
NS = 16
NW = NC * NS

CHUNK = 128
TOT_CHUNKS = E // CHUNK
N_PAD = 10112
RPT = N_PAD // NS

CPT_A = 140
CPT_REM = 4
CPT_B = 16
_C1_BASE = NS * CPT_A + CPT_REM

DEG_CPT = TOT_CHUNKS // NW
DEG_REM = TOT_CHUNKS % NW
DEG_MAX = (DEG_CPT + 1) * CHUNK

_mesh = plsc.VectorSubcoreMesh(core_axis_name="c", subcore_axis_name="s")
_sc_params = pltpu.CompilerParams(needs_layout_passes=False)


@functools.partial(
    pl.kernel,
    out_type=jax.ShapeDtypeStruct((NW, N_PAD), jnp.float32),
    mesh=_mesh,
    compiler_params=_sc_params,
    scratch_types=[
        pltpu.VMEM((DEG_MAX,), jnp.int32),
        pltpu.VMEM((N_PAD,), jnp.float32),
    ],
)
def _deg_kernel(ei_hbm, out_hbm, dstv, degv):
    wid = lax.axis_index("s") * NC + lax.axis_index("c")
    dgo = wid * DEG_CPT + jnp.minimum(wid, DEG_REM)
    dgn = DEG_CPT + (wid < DEG_REM).astype(jnp.int32)
    pltpu.sync_copy(ei_hbm.at[1, pl.ds(dgo * CHUNK, DEG_CPT * CHUNK)],
                    dstv.at[pl.ds(0, DEG_CPT * CHUNK)])

    @pl.when(wid < DEG_REM)
    def _():
        pltpu.sync_copy(ei_hbm.at[1, pl.ds((dgo + DEG_CPT) * CHUNK, CHUNK)],
                        dstv.at[pl.ds(DEG_CPT * CHUNK, CHUNK)])

    zeros16 = jnp.zeros((16,), jnp.float32)
    ones16 = jnp.ones((16,), jnp.float32)

    def zero_body(i, _):
        degv[pl.ds(i * 16, 16)] = zeros16
        return 0

    lax.fori_loop(0, N_PAD // 16, zero_body, 0)

    def hist_body(i, _):
        idx = dstv[pl.ds(i * 16, 16)]
        plsc.addupdate_scatter(degv, [idx], ones16)
        return 0

    lax.fori_loop(0, dgn * (CHUNK // 16), hist_body, 0)
    pltpu.sync_copy(degv, out_hbm.at[wid])


@functools.partial(
    pl.kernel,
    out_type=jax.ShapeDtypeStruct((NC, N_PAD, CH), jnp.float32),
    mesh=_mesh,
    compiler_params=_sc_params,
    scratch_types=[
        pltpu.VMEM((4, CHUNK), jnp.int32),
        pltpu.VMEM((4, CHUNK), jnp.int32),
        pltpu.VMEM((3, CHUNK, CH), jnp.float32),
        pltpu.VMEM_SHARED((N_PAD, CH), jnp.float32),
        pltpu.SemaphoreType.DMA,
        pltpu.SemaphoreType.DMA,
        pltpu.SemaphoreType.DMA,
    ],
)
def _scatter_kernel(y_hbm, ei_hbm, acc_hbm,
                    sidx, didx, gbuf, acc_sh, semI, semG, semS):
    c = lax.axis_index("c")
    s = lax.axis_index("s")
    off = jnp.where(c == 0, s * CPT_A + jnp.minimum(s, CPT_REM),
                    _C1_BASE + s * CPT_B)
    nchunks = jnp.where(c == 0, CPT_A + (s < CPT_REM).astype(jnp.int32),
                        CPT_B)

    pltpu.sync_copy(ei_hbm.at[0, pl.ds(off * CHUNK, CHUNK)], sidx.at[0])
    pltpu.sync_copy(ei_hbm.at[1, pl.ds(off * CHUNK, CHUNK)], didx.at[0])
    pltpu.async_copy(y_hbm.at[sidx.at[0]], gbuf.at[0], semG)
    pltpu.async_copy(ei_hbm.at[0, pl.ds((off + 1) * CHUNK, CHUNK)],
                     sidx.at[1], semI)
    pltpu.async_copy(ei_hbm.at[1, pl.ds((off + 1) * CHUNK, CHUNK)],
                     didx.at[1], semI)

    zeros16 = jnp.zeros((16,), jnp.float32)

    def zero_body(i, _):
        gbuf[2, i // (CH // 16), pl.ds((i % (CH // 16)) * 16, 16)] = zeros16
        return 0

    lax.fori_loop(0, CHUNK * CH // 16, zero_body, 0)

    row0 = s * RPT
    pltpu.sync_copy(gbuf.at[2], acc_sh.at[pl.ds(row0, CHUNK)])
    pltpu.sync_copy(gbuf.at[2], acc_sh.at[pl.ds(row0 + CHUNK, CHUNK)])
    pltpu.sync_copy(gbuf.at[2], acc_sh.at[pl.ds(row0 + 2 * CHUNK, CHUNK)])
    pltpu.sync_copy(gbuf.at[2], acc_sh.at[pl.ds(row0 + 3 * CHUNK, CHUNK)])
    pltpu.sync_copy(gbuf.at[2, pl.ds(0, RPT - 4 * CHUNK)],
                    acc_sh.at[pl.ds(row0 + 4 * CHUNK, RPT - 4 * CHUNK)])
    plsc.subcore_barrier()

    def body(j, _):
        g_cur = j % 3
        g_nxt = (j + 1) % 3
        i_cur = j % 4

        @pl.when(j >= 2)
        def _():
            pltpu.make_async_copy(gbuf.at[g_nxt],
                                  acc_sh.at[pl.ds(row0, CHUNK)],
                                  semS).wait()

        @pl.when(j + 1 < nchunks)
        def _():
            i_nxt = (j + 1) % 4
            pltpu.make_async_copy(
                ei_hbm.at[0, pl.ds((off + j + 1) * CHUNK, CHUNK)],
                sidx.at[i_nxt], semI).wait()
            pltpu.make_async_copy(
                ei_hbm.at[1, pl.ds((off + j + 1) * CHUNK, CHUNK)],
                didx.at[i_nxt], semI).wait()
            pltpu.async_copy(y_hbm.at[sidx.at[i_nxt]], gbuf.at[g_nxt], semG)

        @pl.when(j + 2 < nchunks)
        def _():
            i_2 = (j + 2) % 4
            pltpu.async_copy(ei_hbm.at[0, pl.ds((off + j + 2) * CHUNK, CHUNK)],
                             sidx.at[i_2], semI)
            pltpu.async_copy(ei_hbm.at[1, pl.ds((off + j + 2) * CHUNK, CHUNK)],
                             didx.at[i_2], semI)

        pltpu.make_async_copy(y_hbm.at[sidx.at[i_cur]], gbuf.at[g_cur],
                              semG).wait()
        pltpu.async_copy(gbuf.at[g_cur], acc_sh.at[didx.at[i_cur]], semS,
                         add=True)
        return 0

    lax.fori_loop(0, nchunks, body, 0)

    pltpu.make_async_copy(gbuf.at[0], acc_sh.at[pl.ds(row0, CHUNK)],
                          semS).wait()
    pltpu.make_async_copy(gbuf.at[0], acc_sh.at[pl.ds(row0, CHUNK)],
                          semS).wait()
    plsc.subcore_barrier()

    pltpu.sync_copy(acc_sh.at[pl.ds(row0, RPT)],
                    acc_hbm.at[c, pl.ds(row0, RPT)])


def _linear_body(x_ref, w_ref, deg_ref, y_ref):
    deg = jnp.sum(deg_ref[...], axis=0) + 1.0
    dinv = lax.rsqrt(deg)
    xw = jnp.dot(x_ref[...], w_ref[...], preferred_element_type=jnp.float32)
    y_ref[...] = dinv[:, None] * xw


def _combine_body(acc_ref, y_ref, deg_ref, b_ref, o_ref):
    deg = jnp.sum(deg_ref[...], axis=0) + 1.0
    dinv = lax.rsqrt(deg)
    total = acc_ref[0] + acc_ref[1] + y_ref[...]
    o_ref[...] = dinv[:, None] * total + b_ref[...]


_ROWS_BLK = 1024


def kernel(x, edge_index, W, b):
    ei = edge_index.astype(jnp.int32)

    deg_part = _deg_kernel(ei)

    grid = (pl.cdiv(N, _ROWS_BLK),)
    y = pl.pallas_call(
        _linear_body,
        grid=grid,
        in_specs=[
            pl.BlockSpec((_ROWS_BLK, CH), lambda i: (i, 0)),
            pl.BlockSpec((CH, CH), lambda i: (0, 0)),
            pl.BlockSpec((NW, _ROWS_BLK), lambda i: (0, i)),
        ],
        out_specs=pl.BlockSpec((_ROWS_BLK, CH), lambda i: (i, 0)),
        out_shape=jax.ShapeDtypeStruct((N, CH), jnp.float32),
    )(x, W, deg_part)

    acc = _scatter_kernel(y, ei)

    out = pl.pallas_call(
        _combine_body,
        grid=grid,
        in_specs=[
            pl.BlockSpec((NC, _ROWS_BLK, CH), lambda i: (0, i, 0)),
            pl.BlockSpec((_ROWS_BLK, CH), lambda i: (i, 0)),
            pl.BlockSpec((NW, _ROWS_BLK), lambda i: (0, i)),
            pl.BlockSpec((1, CH), lambda i: (0, 0)),
        ],
        out_specs=pl.BlockSpec((_ROWS_BLK, CH), lambda i: (i, 0)),
        out_shape=jax.ShapeDtypeStruct((N, CH), jnp.float32),
    )(acc, y, deg_part, b.reshape(1, CH))

    return out


# speedup vs baseline: 2.2590x; 1.0389x over previous
import functools

import jax
import jax.numpy as jnp
from jax import lax
from jax.experimental import pallas as pl
from jax.experimental.pallas import tpu as pltpu
from jax.experimental.pallas import tpu_sc as plsc

N = 10000
E = 320000
CH = 128

NC = 2
NS = 16
NW = NC * NS

CHUNK = 128
TOT_CHUNKS = E // CHUNK
N_PAD = 10112
RPT = N_PAD // NS

CPT_A = 132
CPT_REM = 4
CPT_B = 24
_C1_BASE = NS * CPT_A + CPT_REM

DEG_CPT = TOT_CHUNKS // NW
DEG_REM = TOT_CHUNKS % NW
DEG_MAX = (DEG_CPT + 1) * CHUNK

_mesh = plsc.VectorSubcoreMesh(core_axis_name="c", subcore_axis_name="s")
_sc_params = pltpu.CompilerParams(needs_layout_passes=False)


@functools.partial(
    pl.kernel,
    out_type=jax.ShapeDtypeStruct((NW, N_PAD), jnp.float32),
    mesh=_mesh,
    compiler_params=_sc_params,
    scratch_types=[
        pltpu.VMEM((DEG_MAX,), jnp.int32),
        pltpu.VMEM((N_PAD,), jnp.float32),
    ],
)
def _deg_kernel(ei_hbm, out_hbm, dstv, degv):
    wid = lax.axis_index("s") * NC + lax.axis_index("c")
    dgo = wid * DEG_CPT + jnp.minimum(wid, DEG_REM)
    dgn = DEG_CPT + (wid < DEG_REM).astype(jnp.int32)
    pltpu.sync_copy(ei_hbm.at[1, pl.ds(dgo * CHUNK, DEG_CPT * CHUNK)],
                    dstv.at[pl.ds(0, DEG_CPT * CHUNK)])

    @pl.when(wid < DEG_REM)
    def _():
        pltpu.sync_copy(ei_hbm.at[1, pl.ds((dgo + DEG_CPT) * CHUNK, CHUNK)],
                        dstv.at[pl.ds(DEG_CPT * CHUNK, CHUNK)])

    zeros16 = jnp.zeros((16,), jnp.float32)
    ones16 = jnp.ones((16,), jnp.float32)

    def zero_body(i, _):
        degv[pl.ds(i * 16, 16)] = zeros16
        return 0

    lax.fori_loop(0, N_PAD // 16, zero_body, 0)

    def hist_body(i, _):
        idx = dstv[pl.ds(i * 16, 16)]
        plsc.addupdate_scatter(degv, [idx], ones16)
        return 0

    lax.fori_loop(0, dgn * (CHUNK // 16), hist_body, 0)
    pltpu.sync_copy(degv, out_hbm.at[wid])


@functools.partial(
    pl.kernel,
    out_type=jax.ShapeDtypeStruct((NC, N_PAD, CH), jnp.float32),
    mesh=_mesh,
    compiler_params=_sc_params,
    scratch_types=[
        pltpu.VMEM((4, CHUNK), jnp.int32),
        pltpu.VMEM((4, CHUNK), jnp.int32),
        pltpu.VMEM((3, CHUNK, CH), jnp.float32),
        pltpu.VMEM_SHARED((N_PAD, CH), jnp.float32),
        pltpu.SemaphoreType.DMA,
        pltpu.SemaphoreType.DMA,
        pltpu.SemaphoreType.DMA,
    ],
)
def _scatter_kernel(y_hbm, ei_hbm, acc_hbm,
                    sidx, didx, gbuf, acc_sh, semI, semG, semS):
    c = lax.axis_index("c")
    s = lax.axis_index("s")
    off = jnp.where(c == 0, s * CPT_A + jnp.minimum(s, CPT_REM),
                    _C1_BASE + s * CPT_B)
    nchunks = jnp.where(c == 0, CPT_A + (s < CPT_REM).astype(jnp.int32),
                        CPT_B)

    pltpu.sync_copy(ei_hbm.at[0, pl.ds(off * CHUNK, CHUNK)], sidx.at[0])
    pltpu.sync_copy(ei_hbm.at[1, pl.ds(off * CHUNK, CHUNK)], didx.at[0])
    pltpu.async_copy(y_hbm.at[sidx.at[0]], gbuf.at[0], semG)
    pltpu.async_copy(ei_hbm.at[0, pl.ds((off + 1) * CHUNK, CHUNK)],
                     sidx.at[1], semI)
    pltpu.async_copy(ei_hbm.at[1, pl.ds((off + 1) * CHUNK, CHUNK)],
                     didx.at[1], semI)

    zeros16 = jnp.zeros((16,), jnp.float32)

    def zero_body(i, _):
        gbuf[2, i // (CH // 16), pl.ds((i % (CH // 16)) * 16, 16)] = zeros16
        return 0

    lax.fori_loop(0, CHUNK * CH // 16, zero_body, 0)

    row0 = s * RPT
    pltpu.sync_copy(gbuf.at[2], acc_sh.at[pl.ds(row0, CHUNK)])
    pltpu.sync_copy(gbuf.at[2], acc_sh.at[pl.ds(row0 + CHUNK, CHUNK)])
    pltpu.sync_copy(gbuf.at[2], acc_sh.at[pl.ds(row0 + 2 * CHUNK, CHUNK)])
    pltpu.sync_copy(gbuf.at[2], acc_sh.at[pl.ds(row0 + 3 * CHUNK, CHUNK)])
    pltpu.sync_copy(gbuf.at[2, pl.ds(0, RPT - 4 * CHUNK)],
                    acc_sh.at[pl.ds(row0 + 4 * CHUNK, RPT - 4 * CHUNK)])
    plsc.subcore_barrier()

    def body(j, _):
        g_cur = j % 3
        g_nxt = (j + 1) % 3
        i_cur = j % 4

        @pl.when(j >= 2)
        def _():
            pltpu.make_async_copy(gbuf.at[g_nxt],
                                  acc_sh.at[pl.ds(row0, CHUNK)],
                                  semS).wait()

        @pl.when(j + 1 < nchunks)
        def _():
            i_nxt = (j + 1) % 4
            pltpu.make_async_copy(
                ei_hbm.at[0, pl.ds((off + j + 1) * CHUNK, CHUNK)],
                sidx.at[i_nxt], semI).wait()
            pltpu.make_async_copy(
                ei_hbm.at[1, pl.ds((off + j + 1) * CHUNK, CHUNK)],
                didx.at[i_nxt], semI).wait()
            pltpu.async_copy(y_hbm.at[sidx.at[i_nxt]], gbuf.at[g_nxt], semG)

        @pl.when(j + 2 < nchunks)
        def _():
            i_2 = (j + 2) % 4
            pltpu.async_copy(ei_hbm.at[0, pl.ds((off + j + 2) * CHUNK, CHUNK)],
                             sidx.at[i_2], semI)
            pltpu.async_copy(ei_hbm.at[1, pl.ds((off + j + 2) * CHUNK, CHUNK)],
                             didx.at[i_2], semI)

        pltpu.make_async_copy(y_hbm.at[sidx.at[i_cur]], gbuf.at[g_cur],
                              semG).wait()
        pltpu.async_copy(gbuf.at[g_cur], acc_sh.at[didx.at[i_cur]], semS,
                         add=True)
        return 0

    lax.fori_loop(0, nchunks, body, 0)

    pltpu.make_async_copy(gbuf.at[0], acc_sh.at[pl.ds(row0, CHUNK)],
                          semS).wait()
    pltpu.make_async_copy(gbuf.at[0], acc_sh.at[pl.ds(row0, CHUNK)],
                          semS).wait()
    plsc.subcore_barrier()

    pltpu.sync_copy(acc_sh.at[pl.ds(row0, RPT)],
                    acc_hbm.at[c, pl.ds(row0, RPT)])


def _linear_body(x_ref, w_ref, deg_ref, y_ref):
    deg = jnp.sum(deg_ref[...], axis=0) + 1.0
    dinv = lax.rsqrt(deg)
    xw = jnp.dot(x_ref[...], w_ref[...], preferred_element_type=jnp.float32)
    y_ref[...] = dinv[:, None] * xw


def _combine_body(acc_ref, y_ref, deg_ref, b_ref, o_ref):
    deg = jnp.sum(deg_ref[...], axis=0) + 1.0
    dinv = lax.rsqrt(deg)
    total = acc_ref[0] + acc_ref[1] + y_ref[...]
    o_ref[...] = dinv[:, None] * total + b_ref[...]


_ROWS_BLK = 1024


def kernel(x, edge_index, W, b):
    ei = edge_index.astype(jnp.int32)

    deg_part = _deg_kernel(ei)

    grid = (pl.cdiv(N, _ROWS_BLK),)
    y = pl.pallas_call(
        _linear_body,
        grid=grid,
        in_specs=[
            pl.BlockSpec((_ROWS_BLK, CH), lambda i: (i, 0)),
            pl.BlockSpec((CH, CH), lambda i: (0, 0)),
            pl.BlockSpec((NW, _ROWS_BLK), lambda i: (0, i)),
        ],
        out_specs=pl.BlockSpec((_ROWS_BLK, CH), lambda i: (i, 0)),
        out_shape=jax.ShapeDtypeStruct((N, CH), jnp.float32),
    )(x, W, deg_part)

    acc = _scatter_kernel(y, ei)

    out = pl.pallas_call(
        _combine_body,
        grid=grid,
        in_specs=[
            pl.BlockSpec((NC, _ROWS_BLK, CH), lambda i: (0, i, 0)),
            pl.BlockSpec((_ROWS_BLK, CH), lambda i: (i, 0)),
            pl.BlockSpec((NW, _ROWS_BLK), lambda i: (0, i)),
            pl.BlockSpec((1, CH), lambda i: (0, 0)),
        ],
        out_specs=pl.BlockSpec((_ROWS_BLK, CH), lambda i: (i, 0)),
        out_shape=jax.ShapeDtypeStruct((N, CH), jnp.float32),
    )(acc, y, deg_part, b.reshape(1, CH))

    return out


# speedup vs baseline: 2.3675x; 1.0480x over previous
import functools

import jax
import jax.numpy as jnp
from jax import lax
from jax.experimental import pallas as pl
from jax.experimental.pallas import tpu as pltpu
from jax.experimental.pallas import tpu_sc as plsc

N = 10000
E = 320000
CH = 128

NC = 2
NS = 16
NW = NC * NS

CHUNK = 128
TOT_CHUNKS = E // CHUNK
N_PAD = 10112
RPT = N_PAD // NS

CPT_A = 124
CPT_REM = 4
CPT_B = 32
_C1_BASE = NS * CPT_A + CPT_REM

DEG_CPT = TOT_CHUNKS // NW
DEG_REM = TOT_CHUNKS % NW
DEG_MAX = (DEG_CPT + 1) * CHUNK

_mesh = plsc.VectorSubcoreMesh(core_axis_name="c", subcore_axis_name="s")
_sc_params = pltpu.CompilerParams(needs_layout_passes=False)


@functools.partial(
    pl.kernel,
    out_type=jax.ShapeDtypeStruct((NW, N_PAD), jnp.float32),
    mesh=_mesh,
    compiler_params=_sc_params,
    scratch_types=[
        pltpu.VMEM((DEG_MAX,), jnp.int32),
        pltpu.VMEM((N_PAD,), jnp.float32),
    ],
)
def _deg_kernel(ei_hbm, out_hbm, dstv, degv):
    wid = lax.axis_index("s") * NC + lax.axis_index("c")
    dgo = wid * DEG_CPT + jnp.minimum(wid, DEG_REM)
    dgn = DEG_CPT + (wid < DEG_REM).astype(jnp.int32)
    pltpu.sync_copy(ei_hbm.at[1, pl.ds(dgo * CHUNK, DEG_CPT * CHUNK)],
                    dstv.at[pl.ds(0, DEG_CPT * CHUNK)])

    @pl.when(wid < DEG_REM)
    def _():
        pltpu.sync_copy(ei_hbm.at[1, pl.ds((dgo + DEG_CPT) * CHUNK, CHUNK)],
                        dstv.at[pl.ds(DEG_CPT * CHUNK, CHUNK)])

    zeros16 = jnp.zeros((16,), jnp.float32)
    ones16 = jnp.ones((16,), jnp.float32)

    def zero_body(i, _):
        degv[pl.ds(i * 16, 16)] = zeros16
        return 0

    lax.fori_loop(0, N_PAD // 16, zero_body, 0)

    def hist_body(i, _):
        idx = dstv[pl.ds(i * 16, 16)]
        plsc.addupdate_scatter(degv, [idx], ones16)
        return 0

    lax.fori_loop(0, dgn * (CHUNK // 16), hist_body, 0)
    pltpu.sync_copy(degv, out_hbm.at[wid])


@functools.partial(
    pl.kernel,
    out_type=jax.ShapeDtypeStruct((NC, N_PAD, CH), jnp.float32),
    mesh=_mesh,
    compiler_params=_sc_params,
    scratch_types=[
        pltpu.VMEM((4, CHUNK), jnp.int32),
        pltpu.VMEM((4, CHUNK), jnp.int32),
        pltpu.VMEM((3, CHUNK, CH), jnp.float32),
        pltpu.VMEM_SHARED((N_PAD, CH), jnp.float32),
        pltpu.SemaphoreType.DMA,
        pltpu.SemaphoreType.DMA,
        pltpu.SemaphoreType.DMA,
    ],
)
def _scatter_kernel(y_hbm, ei_hbm, acc_hbm,
                    sidx, didx, gbuf, acc_sh, semI, semG, semS):
    c = lax.axis_index("c")
    s = lax.axis_index("s")
    off = jnp.where(c == 0, s * CPT_A + jnp.minimum(s, CPT_REM),
                    _C1_BASE + s * CPT_B)
    nchunks = jnp.where(c == 0, CPT_A + (s < CPT_REM).astype(jnp.int32),
                        CPT_B)

    pltpu.sync_copy(ei_hbm.at[0, pl.ds(off * CHUNK, CHUNK)], sidx.at[0])
    pltpu.sync_copy(ei_hbm.at[1, pl.ds(off * CHUNK, CHUNK)], didx.at[0])
    pltpu.async_copy(y_hbm.at[sidx.at[0]], gbuf.at[0], semG)
    pltpu.async_copy(ei_hbm.at[0, pl.ds((off + 1) * CHUNK, CHUNK)],
                     sidx.at[1], semI)
    pltpu.async_copy(ei_hbm.at[1, pl.ds((off + 1) * CHUNK, CHUNK)],
                     didx.at[1], semI)

    zeros16 = jnp.zeros((16,), jnp.float32)

    def zero_body(i, _):
        gbuf[2, i // (CH // 16), pl.ds((i % (CH // 16)) * 16, 16)] = zeros16
        return 0

    lax.fori_loop(0, CHUNK * CH // 16, zero_body, 0)

    row0 = s * RPT
    pltpu.sync_copy(gbuf.at[2], acc_sh.at[pl.ds(row0, CHUNK)])
    pltpu.sync_copy(gbuf.at[2], acc_sh.at[pl.ds(row0 + CHUNK, CHUNK)])
    pltpu.sync_copy(gbuf.at[2], acc_sh.at[pl.ds(row0 + 2 * CHUNK, CHUNK)])
    pltpu.sync_copy(gbuf.at[2], acc_sh.at[pl.ds(row0 + 3 * CHUNK, CHUNK)])
    pltpu.sync_copy(gbuf.at[2, pl.ds(0, RPT - 4 * CHUNK)],
                    acc_sh.at[pl.ds(row0 + 4 * CHUNK, RPT - 4 * CHUNK)])
    plsc.subcore_barrier()

    def body(j, _):
        g_cur = j % 3
        g_nxt = (j + 1) % 3
        i_cur = j % 4

        @pl.when(j >= 2)
        def _():
            pltpu.make_async_copy(gbuf.at[g_nxt],
                                  acc_sh.at[pl.ds(row0, CHUNK)],
                                  semS).wait()

        @pl.when(j + 1 < nchunks)
        def _():
            i_nxt = (j + 1) % 4
            pltpu.make_async_copy(
                ei_hbm.at[0, pl.ds((off + j + 1) * CHUNK, CHUNK)],
                sidx.at[i_nxt], semI).wait()
            pltpu.make_async_copy(
                ei_hbm.at[1, pl.ds((off + j + 1) * CHUNK, CHUNK)],
                didx.at[i_nxt], semI).wait()
            pltpu.async_copy(y_hbm.at[sidx.at[i_nxt]], gbuf.at[g_nxt], semG)

        @pl.when(j + 2 < nchunks)
        def _():
            i_2 = (j + 2) % 4
            pltpu.async_copy(ei_hbm.at[0, pl.ds((off + j + 2) * CHUNK, CHUNK)],
                             sidx.at[i_2], semI)
            pltpu.async_copy(ei_hbm.at[1, pl.ds((off + j + 2) * CHUNK, CHUNK)],
                             didx.at[i_2], semI)

        pltpu.make_async_copy(y_hbm.at[sidx.at[i_cur]], gbuf.at[g_cur],
                              semG).wait()
        pltpu.async_copy(gbuf.at[g_cur], acc_sh.at[didx.at[i_cur]], semS,
                         add=True)
        return 0

    lax.fori_loop(0, nchunks, body, 0)

    pltpu.make_async_copy(gbuf.at[0], acc_sh.at[pl.ds(row0, CHUNK)],
                          semS).wait()
    pltpu.make_async_copy(gbuf.at[0], acc_sh.at[pl.ds(row0, CHUNK)],
                          semS).wait()
    plsc.subcore_barrier()

    pltpu.sync_copy(acc_sh.at[pl.ds(row0, RPT)],
                    acc_hbm.at[c, pl.ds(row0, RPT)])


def _linear_body(x_ref, w_ref, deg_ref, y_ref):
    deg = jnp.sum(deg_ref[...], axis=0) + 1.0
    dinv = lax.rsqrt(deg)
    xw = jnp.dot(x_ref[...], w_ref[...], preferred_element_type=jnp.float32)
    y_ref[...] = dinv[:, None] * xw


def _combine_body(acc_ref, y_ref, deg_ref, b_ref, o_ref):
    deg = jnp.sum(deg_ref[...], axis=0) + 1.0
    dinv = lax.rsqrt(deg)
    total = acc_ref[0] + acc_ref[1] + y_ref[...]
    o_ref[...] = dinv[:, None] * total + b_ref[...]


_ROWS_BLK = 1024


def kernel(x, edge_index, W, b):
    ei = edge_index.astype(jnp.int32)

    deg_part = _deg_kernel(ei)

    grid = (pl.cdiv(N, _ROWS_BLK),)
    y = pl.pallas_call(
        _linear_body,
        grid=grid,
        in_specs=[
            pl.BlockSpec((_ROWS_BLK, CH), lambda i: (i, 0)),
            pl.BlockSpec((CH, CH), lambda i: (0, 0)),
            pl.BlockSpec((NW, _ROWS_BLK), lambda i: (0, i)),
        ],
        out_specs=pl.BlockSpec((_ROWS_BLK, CH), lambda i: (i, 0)),
        out_shape=jax.ShapeDtypeStruct((N, CH), jnp.float32),
    )(x, W, deg_part)

    acc = _scatter_kernel(y, ei)

    out = pl.pallas_call(
        _combine_body,
        grid=grid,
        in_specs=[
            pl.BlockSpec((NC, _ROWS_BLK, CH), lambda i: (0, i, 0)),
            pl.BlockSpec((_ROWS_BLK, CH), lambda i: (i, 0)),
            pl.BlockSpec((NW, _ROWS_BLK), lambda i: (0, i)),
            pl.BlockSpec((1, CH), lambda i: (0, 0)),
        ],
        out_specs=pl.BlockSpec((_ROWS_BLK, CH), lambda i: (i, 0)),
        out_shape=jax.ShapeDtypeStruct((N, CH), jnp.float32),
    )(acc, y, deg_part, b.reshape(1, CH))

    return out


# speedup vs baseline: 2.4676x; 1.0423x over previous
import functools

import jax
import jax.numpy as jnp
from jax import lax
from jax.experimental import pallas as pl
from jax.experimental.pallas import tpu as pltpu
from jax.experimental.pallas import tpu_sc as plsc

N = 10000
E = 320000
CH = 128

NC = 2
NS = 16
NW = NC * NS

CHUNK = 128
TOT_CHUNKS = E // CHUNK
N_PAD = 10112
RPT = N_PAD // NS

CPT_A = 116
CPT_REM = 4
CPT_B = 40
_C1_BASE = NS * CPT_A + CPT_REM

DEG_CPT = TOT_CHUNKS // NW
DEG_REM = TOT_CHUNKS % NW
DEG_MAX = (DEG_CPT + 1) * CHUNK

_mesh = plsc.VectorSubcoreMesh(core_axis_name="c", subcore_axis_name="s")
_sc_params = pltpu.CompilerParams(needs_layout_passes=False)


@functools.partial(
    pl.kernel,
    out_type=jax.ShapeDtypeStruct((NW, N_PAD), jnp.float32),
    mesh=_mesh,
    compiler_params=_sc_params,
    scratch_types=[
        pltpu.VMEM((DEG_MAX,), jnp.int32),
        pltpu.VMEM((N_PAD,), jnp.float32),
    ],
)
def _deg_kernel(ei_hbm, out_hbm, dstv, degv):
    wid = lax.axis_index("s") * NC + lax.axis_index("c")
    dgo = wid * DEG_CPT + jnp.minimum(wid, DEG_REM)
    dgn = DEG_CPT + (wid < DEG_REM).astype(jnp.int32)
    pltpu.sync_copy(ei_hbm.at[1, pl.ds(dgo * CHUNK, DEG_CPT * CHUNK)],
                    dstv.at[pl.ds(0, DEG_CPT * CHUNK)])

    @pl.when(wid < DEG_REM)
    def _():
        pltpu.sync_copy(ei_hbm.at[1, pl.ds((dgo + DEG_CPT) * CHUNK, CHUNK)],
                        dstv.at[pl.ds(DEG_CPT * CHUNK, CHUNK)])

    zeros16 = jnp.zeros((16,), jnp.float32)
    ones16 = jnp.ones((16,), jnp.float32)

    def zero_body(i, _):
        degv[pl.ds(i * 16, 16)] = zeros16
        return 0

    lax.fori_loop(0, N_PAD // 16, zero_body, 0)

    def hist_body(i, _):
        idx = dstv[pl.ds(i * 16, 16)]
        plsc.addupdate_scatter(degv, [idx], ones16)
        return 0

    lax.fori_loop(0, dgn * (CHUNK // 16), hist_body, 0)
    pltpu.sync_copy(degv, out_hbm.at[wid])


@functools.partial(
    pl.kernel,
    out_type=jax.ShapeDtypeStruct((NC, N_PAD, CH), jnp.float32),
    mesh=_mesh,
    compiler_params=_sc_params,
    scratch_types=[
        pltpu.VMEM((4, CHUNK), jnp.int32),
        pltpu.VMEM((4, CHUNK), jnp.int32),
        pltpu.VMEM((3, CHUNK, CH), jnp.float32),
        pltpu.VMEM_SHARED((N_PAD, CH), jnp.float32),
        pltpu.SemaphoreType.DMA,
        pltpu.SemaphoreType.DMA,
        pltpu.SemaphoreType.DMA,
    ],
)
def _scatter_kernel(y_hbm, ei_hbm, acc_hbm,
                    sidx, didx, gbuf, acc_sh, semI, semG, semS):
    c = lax.axis_index("c")
    s = lax.axis_index("s")
    off = jnp.where(c == 0, s * CPT_A + jnp.minimum(s, CPT_REM),
                    _C1_BASE + s * CPT_B)
    nchunks = jnp.where(c == 0, CPT_A + (s < CPT_REM).astype(jnp.int32),
                        CPT_B)

    pltpu.sync_copy(ei_hbm.at[0, pl.ds(off * CHUNK, CHUNK)], sidx.at[0])
    pltpu.sync_copy(ei_hbm.at[1, pl.ds(off * CHUNK, CHUNK)], didx.at[0])
    pltpu.async_copy(y_hbm.at[sidx.at[0]], gbuf.at[0], semG)
    pltpu.async_copy(ei_hbm.at[0, pl.ds((off + 1) * CHUNK, CHUNK)],
                     sidx.at[1], semI)
    pltpu.async_copy(ei_hbm.at[1, pl.ds((off + 1) * CHUNK, CHUNK)],
                     didx.at[1], semI)

    zeros16 = jnp.zeros((16,), jnp.float32)

    def zero_body(i, _):
        gbuf[2, i // (CH // 16), pl.ds((i % (CH // 16)) * 16, 16)] = zeros16
        return 0

    lax.fori_loop(0, CHUNK * CH // 16, zero_body, 0)

    row0 = s * RPT
    pltpu.sync_copy(gbuf.at[2], acc_sh.at[pl.ds(row0, CHUNK)])
    pltpu.sync_copy(gbuf.at[2], acc_sh.at[pl.ds(row0 + CHUNK, CHUNK)])
    pltpu.sync_copy(gbuf.at[2], acc_sh.at[pl.ds(row0 + 2 * CHUNK, CHUNK)])
    pltpu.sync_copy(gbuf.at[2], acc_sh.at[pl.ds(row0 + 3 * CHUNK, CHUNK)])
    pltpu.sync_copy(gbuf.at[2, pl.ds(0, RPT - 4 * CHUNK)],
                    acc_sh.at[pl.ds(row0 + 4 * CHUNK, RPT - 4 * CHUNK)])
    plsc.subcore_barrier()

    def body(j, _):
        g_cur = j % 3
        g_nxt = (j + 1) % 3
        i_cur = j % 4

        @pl.when(j >= 2)
        def _():
            pltpu.make_async_copy(gbuf.at[g_nxt],
                                  acc_sh.at[pl.ds(row0, CHUNK)],
                                  semS).wait()

        @pl.when(j + 1 < nchunks)
        def _():
            i_nxt = (j + 1) % 4
            pltpu.make_async_copy(
                ei_hbm.at[0, pl.ds((off + j + 1) * CHUNK, CHUNK)],
                sidx.at[i_nxt], semI).wait()
            pltpu.make_async_copy(
                ei_hbm.at[1, pl.ds((off + j + 1) * CHUNK, CHUNK)],
                didx.at[i_nxt], semI).wait()
            pltpu.async_copy(y_hbm.at[sidx.at[i_nxt]], gbuf.at[g_nxt], semG)

        @pl.when(j + 2 < nchunks)
        def _():
            i_2 = (j + 2) % 4
            pltpu.async_copy(ei_hbm.at[0, pl.ds((off + j + 2) * CHUNK, CHUNK)],
                             sidx.at[i_2], semI)
            pltpu.async_copy(ei_hbm.at[1, pl.ds((off + j + 2) * CHUNK, CHUNK)],
                             didx.at[i_2], semI)

        pltpu.make_async_copy(y_hbm.at[sidx.at[i_cur]], gbuf.at[g_cur],
                              semG).wait()
        pltpu.async_copy(gbuf.at[g_cur], acc_sh.at[didx.at[i_cur]], semS,
                         add=True)
        return 0

    lax.fori_loop(0, nchunks, body, 0)

    pltpu.make_async_copy(gbuf.at[0], acc_sh.at[pl.ds(row0, CHUNK)],
                          semS).wait()
    pltpu.make_async_copy(gbuf.at[0], acc_sh.at[pl.ds(row0, CHUNK)],
                          semS).wait()
    plsc.subcore_barrier()

    pltpu.sync_copy(acc_sh.at[pl.ds(row0, RPT)],
                    acc_hbm.at[c, pl.ds(row0, RPT)])


def _linear_body(x_ref, w_ref, deg_ref, y_ref):
    deg = jnp.sum(deg_ref[...], axis=0) + 1.0
    dinv = lax.rsqrt(deg)
    xw = jnp.dot(x_ref[...], w_ref[...], preferred_element_type=jnp.float32)
    y_ref[...] = dinv[:, None] * xw


def _combine_body(acc_ref, y_ref, deg_ref, b_ref, o_ref):
    deg = jnp.sum(deg_ref[...], axis=0) + 1.0
    dinv = lax.rsqrt(deg)
    total = acc_ref[0] + acc_ref[1] + y_ref[...]
    o_ref[...] = dinv[:, None] * total + b_ref[...]


_ROWS_BLK = 1024


def kernel(x, edge_index, W, b):
    ei = edge_index.astype(jnp.int32)

    deg_part = _deg_kernel(ei)

    grid = (pl.cdiv(N, _ROWS_BLK),)
    y = pl.pallas_call(
        _linear_body,
        grid=grid,
        in_specs=[
            pl.BlockSpec((_ROWS_BLK, CH), lambda i: (i, 0)),
            pl.BlockSpec((CH, CH), lambda i: (0, 0)),
            pl.BlockSpec((NW, _ROWS_BLK), lambda i: (0, i)),
        ],
        out_specs=pl.BlockSpec((_ROWS_BLK, CH), lambda i: (i, 0)),
        out_shape=jax.ShapeDtypeStruct((N, CH), jnp.float32),
    )(x, W, deg_part)

    acc = _scatter_kernel(y, ei)

    out = pl.pallas_call(
        _combine_body,
        grid=grid,
        in_specs=[
            pl.BlockSpec((NC, _ROWS_BLK, CH), lambda i: (0, i, 0)),
            pl.BlockSpec((_ROWS_BLK, CH), lambda i: (i, 0)),
            pl.BlockSpec((NW, _ROWS_BLK), lambda i: (0, i)),
            pl.BlockSpec((1, CH), lambda i: (0, 0)),
        ],
        out_specs=pl.BlockSpec((_ROWS_BLK, CH), lambda i: (i, 0)),
        out_shape=jax.ShapeDtypeStruct((N, CH), jnp.float32),
    )(acc, y, deg_part, b.reshape(1, CH))

    return out


# speedup vs baseline: 3.0955x; 1.2544x over previous
import functools

import jax
import jax.numpy as jnp
from jax import lax
from jax.experimental import pallas as pl
from jax.experimental.pallas import tpu as pltpu
from jax.experimental.pallas import tpu_sc as plsc

N = 10000
E = 320000
CH = 128

NC = 2
NS = 16
NW = NC * NS

CHUNK = 128
TOT_CHUNKS = E // CHUNK
N_PAD = 10112
RPT = N_PAD // NS

CPT_A = 78
CPT_REM = 4
CPT_B = 78
_C1_BASE = NS * CPT_A + CPT_REM

DEG_CPT = TOT_CHUNKS // NW
DEG_REM = TOT_CHUNKS % NW
DEG_MAX = (DEG_CPT + 1) * CHUNK

_mesh = plsc.VectorSubcoreMesh(core_axis_name="c", subcore_axis_name="s")
_sc_params = pltpu.CompilerParams(needs_layout_passes=False)


@functools.partial(
    pl.kernel,
    out_type=jax.ShapeDtypeStruct((NW, N_PAD), jnp.float32),
    mesh=_mesh,
    compiler_params=_sc_params,
    scratch_types=[
        pltpu.VMEM((DEG_MAX,), jnp.int32),
        pltpu.VMEM((N_PAD,), jnp.float32),
    ],
)
def _deg_kernel(ei_hbm, out_hbm, dstv, degv):
    wid = lax.axis_index("s") * NC + lax.axis_index("c")
    dgo = wid * DEG_CPT + jnp.minimum(wid, DEG_REM)
    dgn = DEG_CPT + (wid < DEG_REM).astype(jnp.int32)
    pltpu.sync_copy(ei_hbm.at[1, pl.ds(dgo * CHUNK, DEG_CPT * CHUNK)],
                    dstv.at[pl.ds(0, DEG_CPT * CHUNK)])

    @pl.when(wid < DEG_REM)
    def _():
        pltpu.sync_copy(ei_hbm.at[1, pl.ds((dgo + DEG_CPT) * CHUNK, CHUNK)],
                        dstv.at[pl.ds(DEG_CPT * CHUNK, CHUNK)])

    zeros16 = jnp.zeros((16,), jnp.float32)
    ones16 = jnp.ones((16,), jnp.float32)

    def zero_body(i, _):
        degv[pl.ds(i * 16, 16)] = zeros16
        return 0

    lax.fori_loop(0, N_PAD // 16, zero_body, 0)

    def hist_body(i, _):
        idx = dstv[pl.ds(i * 16, 16)]
        plsc.addupdate_scatter(degv, [idx], ones16)
        return 0

    lax.fori_loop(0, dgn * (CHUNK // 16), hist_body, 0)
    pltpu.sync_copy(degv, out_hbm.at[wid])


@functools.partial(
    pl.kernel,
    out_type=jax.ShapeDtypeStruct((NC, N_PAD, CH), jnp.float32),
    mesh=_mesh,
    compiler_params=_sc_params,
    scratch_types=[
        pltpu.VMEM((4, CHUNK), jnp.int32),
        pltpu.VMEM((4, CHUNK), jnp.int32),
        pltpu.VMEM((3, CHUNK, CH), jnp.float32),
        pltpu.VMEM_SHARED((N_PAD, CH), jnp.float32),
        pltpu.SemaphoreType.DMA,
        pltpu.SemaphoreType.DMA,
        pltpu.SemaphoreType.DMA,
    ],
)
def _scatter_kernel(y_hbm, ei_hbm, acc_hbm,
                    sidx, didx, gbuf, acc_sh, semI, semG, semS):
    c = lax.axis_index("c")
    s = lax.axis_index("s")
    off = jnp.where(c == 0, s * CPT_A + jnp.minimum(s, CPT_REM),
                    _C1_BASE + s * CPT_B)
    nchunks = jnp.where(c == 0, CPT_A + (s < CPT_REM).astype(jnp.int32),
                        CPT_B)

    pltpu.sync_copy(ei_hbm.at[0, pl.ds(off * CHUNK, CHUNK)], sidx.at[0])
    pltpu.sync_copy(ei_hbm.at[1, pl.ds(off * CHUNK, CHUNK)], didx.at[0])
    pltpu.async_copy(y_hbm.at[sidx.at[0]], gbuf.at[0], semG)
    pltpu.async_copy(ei_hbm.at[0, pl.ds((off + 1) * CHUNK, CHUNK)],
                     sidx.at[1], semI)
    pltpu.async_copy(ei_hbm.at[1, pl.ds((off + 1) * CHUNK, CHUNK)],
                     didx.at[1], semI)

    zeros16 = jnp.zeros((16,), jnp.float32)

    def zero_body(i, _):
        gbuf[2, i // (CH // 16), pl.ds((i % (CH // 16)) * 16, 16)] = zeros16
        return 0

    lax.fori_loop(0, CHUNK * CH // 16, zero_body, 0)

    row0 = s * RPT
    pltpu.sync_copy(gbuf.at[2], acc_sh.at[pl.ds(row0, CHUNK)])
    pltpu.sync_copy(gbuf.at[2], acc_sh.at[pl.ds(row0 + CHUNK, CHUNK)])
    pltpu.sync_copy(gbuf.at[2], acc_sh.at[pl.ds(row0 + 2 * CHUNK, CHUNK)])
    pltpu.sync_copy(gbuf.at[2], acc_sh.at[pl.ds(row0 + 3 * CHUNK, CHUNK)])
    pltpu.sync_copy(gbuf.at[2, pl.ds(0, RPT - 4 * CHUNK)],
                    acc_sh.at[pl.ds(row0 + 4 * CHUNK, RPT - 4 * CHUNK)])
    plsc.subcore_barrier()

    def body(j, _):
        g_cur = j % 3
        g_nxt = (j + 1) % 3
        i_cur = j % 4

        @pl.when(j >= 2)
        def _():
            pltpu.make_async_copy(gbuf.at[g_nxt],
                                  acc_sh.at[pl.ds(row0, CHUNK)],
                                  semS).wait()

        @pl.when(j + 1 < nchunks)
        def _():
            i_nxt = (j + 1) % 4
            pltpu.make_async_copy(
                ei_hbm.at[0, pl.ds((off + j + 1) * CHUNK, CHUNK)],
                sidx.at[i_nxt], semI).wait()
            pltpu.make_async_copy(
                ei_hbm.at[1, pl.ds((off + j + 1) * CHUNK, CHUNK)],
                didx.at[i_nxt], semI).wait()
            pltpu.async_copy(y_hbm.at[sidx.at[i_nxt]], gbuf.at[g_nxt], semG)

        @pl.when(j + 2 < nchunks)
        def _():
            i_2 = (j + 2) % 4
            pltpu.async_copy(ei_hbm.at[0, pl.ds((off + j + 2) * CHUNK, CHUNK)],
                             sidx.at[i_2], semI)
            pltpu.async_copy(ei_hbm.at[1, pl.ds((off + j + 2) * CHUNK, CHUNK)],
                             didx.at[i_2], semI)

        pltpu.make_async_copy(y_hbm.at[sidx.at[i_cur]], gbuf.at[g_cur],
                              semG).wait()
        pltpu.async_copy(gbuf.at[g_cur], acc_sh.at[didx.at[i_cur]], semS,
                         add=True)
        return 0

    lax.fori_loop(0, nchunks, body, 0)

    pltpu.make_async_copy(gbuf.at[0], acc_sh.at[pl.ds(row0, CHUNK)],
                          semS).wait()
    pltpu.make_async_copy(gbuf.at[0], acc_sh.at[pl.ds(row0, CHUNK)],
                          semS).wait()
    plsc.subcore_barrier()

    pltpu.sync_copy(acc_sh.at[pl.ds(row0, RPT)],
                    acc_hbm.at[c, pl.ds(row0, RPT)])


def _linear_body(x_ref, w_ref, deg_ref, y_ref):
    deg = jnp.sum(deg_ref[...], axis=0) + 1.0
    dinv = lax.rsqrt(deg)
    xw = jnp.dot(x_ref[...], w_ref[...], preferred_element_type=jnp.float32)
    y_ref[...] = dinv[:, None] * xw


def _combine_body(acc_ref, y_ref, deg_ref, b_ref, o_ref):
    deg = jnp.sum(deg_ref[...], axis=0) + 1.0
    dinv = lax.rsqrt(deg)
    total = acc_ref[0] + acc_ref[1] + y_ref[...]
    o_ref[...] = dinv[:, None] * total + b_ref[...]


_ROWS_BLK = 1024


def kernel(x, edge_index, W, b):
    ei = edge_index.astype(jnp.int32)

    deg_part = _deg_kernel(ei)

    grid = (pl.cdiv(N, _ROWS_BLK),)
    y = pl.pallas_call(
        _linear_body,
        grid=grid,
        in_specs=[
            pl.BlockSpec((_ROWS_BLK, CH), lambda i: (i, 0)),
            pl.BlockSpec((CH, CH), lambda i: (0, 0)),
            pl.BlockSpec((NW, _ROWS_BLK), lambda i: (0, i)),
        ],
        out_specs=pl.BlockSpec((_ROWS_BLK, CH), lambda i: (i, 0)),
        out_shape=jax.ShapeDtypeStruct((N, CH), jnp.float32),
    )(x, W, deg_part)

    acc = _scatter_kernel(y, ei)

    out = pl.pallas_call(
        _combine_body,
        grid=grid,
        in_specs=[
            pl.BlockSpec((NC, _ROWS_BLK, CH), lambda i: (0, i, 0)),
            pl.BlockSpec((_ROWS_BLK, CH), lambda i: (i, 0)),
            pl.BlockSpec((NW, _ROWS_BLK), lambda i: (0, i)),
            pl.BlockSpec((1, CH), lambda i: (0, 0)),
        ],
        out_specs=pl.BlockSpec((_ROWS_BLK, CH), lambda i: (i, 0)),
        out_shape=jax.ShapeDtypeStruct((N, CH), jnp.float32),
    )(acc, y, deg_part, b.reshape(1, CH))

    return out
